# baseline (speedup 1.0000x reference)
import jax
import jax.numpy as jnp
from jax.experimental import pallas as pl

L = 4
N = 10000


def _layer_norm(v, g, b):
    mu = jnp.mean(v, axis=-1, keepdims=True)
    var = jnp.var(v, axis=-1, keepdims=True)
    return (v - mu) / jnp.sqrt(var + 1e-5) * g + b


def kernel(x, e, edge_index, params):
    src = edge_index[0]
    dst = edge_index[1]
    h = jnp.maximum(x @ params['ne_W1'] + params['ne_b1'], 0.0) @ params['ne_W2'] + params['ne_b2']
    e_h = jnp.maximum(e @ params['ee_W1'] + params['ee_b1'], 0.0) @ params['ee_W2'] + params['ee_b2']
    e = e_h
    for i in range(L):
        h_in, e_in = h, e
        A1h = h @ params['A1_W'][i] + params['A1_b'][i]
        A2h = h @ params['A2_W'][i] + params['A2_b'][i]
        A3h = h @ params['A3_W'][i] + params['A3_b'][i]
        B1h = h @ params['B1_W'][i] + params['B1_b'][i]
        B2h = h @ params['B2_W'][i] + params['B2_b'][i]
        B3e = e @ params['B3_W'][i] + params['B3_b'][i]
        e_hat = B1h[src] + B2h[dst] + B3e
        e_hat = jnp.maximum(_layer_norm(e_hat, params['ln_e_g'][i], params['ln_e_b'][i]), 0.0)
        e = e_in + e_hat
        sigma = jax.nn.sigmoid(e)
        num_f = jax.ops.segment_sum(sigma * A2h[src], dst, num_segments=N)
        den_f = jax.ops.segment_sum(sigma, dst, num_segments=N)
        h_f = num_f / (den_f + 1e-6)
        num_b = jax.ops.segment_sum(sigma * A3h[dst], src, num_segments=N)
        den_b = jax.ops.segment_sum(sigma, src, num_segments=N)
        h_b = num_b / (den_b + 1e-6)
        h_hat = A1h + h_f + h_b
        h_hat = jnp.maximum(_layer_norm(h_hat, params['ln_h_g'][i], params['ln_h_b'][i]), 0.0)
        h = h_in + h_hat
    data = jnp.concatenate([h[src], h[dst], e], axis=1)
    z = jnp.maximum(data @ params['pred_W1'] + params['pred_b1'], 0.0)
    scores = z @ params['pred_W2'] + params['pred_b2']
    return scores


# speedup vs baseline: 2.8321x; 2.8321x over previous
import functools

import jax
import jax.numpy as jnp
from jax import lax
from jax.experimental import pallas as pl
from jax.experimental.pallas import tpu as pltpu
from jax.experimental.pallas import tpu_sc as plsc

NN = 10000
EE = 320000
HID = 128
NCH = 2
CH = 64
LAYERS = 4
SC_CORES = 2
SC_TILES = 16

EBT = 2000


def _mesh():
    return plsc.VectorSubcoreMesh(
        core_axis_name="c", subcore_axis_name="s",
        num_cores=SC_CORES, num_subcores=SC_TILES)


def _sc_gather2add(ta, tb, src1d, dst1d, D):
    EB = 256
    G2 = EB // 128

    @functools.partial(
        pl.kernel,
        out_type=jax.ShapeDtypeStruct((EE, D), jnp.float32),
        mesh=_mesh(),
        scratch_types=[
            pltpu.VMEM((EB,), jnp.int32),
            pltpu.VMEM((EB,), jnp.int32),
            pltpu.VMEM((G2, 128), jnp.int32),
            pltpu.VMEM((G2, 128), jnp.int32),
            pltpu.VMEM((EB, D), jnp.float32),
            pltpu.VMEM((EB, D), jnp.float32),
            pltpu.SemaphoreType.DMA,
        ],
    )
    def k(ta_ref, tb_ref, s_ref, d_ref, out_ref,
          ids1, idd1, ids, idd, abuf, bbuf, sem):
        w = lax.axis_index("s") * SC_CORES + lax.axis_index("c")
        nblk = jnp.where(w == 31, 10, 40)

        def body(b, carry):
            base = w * 10240 + b * EB
            pltpu.sync_copy(s_ref.at[pl.ds(base, EB)], ids1)
            pltpu.sync_copy(d_ref.at[pl.ds(base, EB)], idd1)
            for g in range(G2):
                for j in range(8):
                    jl = pl.ds(j * 16, 16)
                    fl = pl.ds(g * 128 + j * 16, 16)
                    ids[g, jl] = ids1[fl]
                    idd[g, jl] = idd1[fl]
            cps = []
            for g in range(G2):
                sl = pl.ds(g * 128, 128)
                cps.append(pltpu.async_copy(ta_ref.at[ids.at[g]], abuf.at[sl], sem))
                cps.append(pltpu.async_copy(tb_ref.at[idd.at[g]], bbuf.at[sl], sem))
            for cp in cps:
                cp.wait()

            def add_row(r, c):
                for j in range(D // 16):
                    jl = pl.ds(j * 16, 16)
                    abuf[r, jl] = abuf[r, jl] + bbuf[r, jl]
                return c

            lax.fori_loop(0, EB, add_row, 0)
            pltpu.sync_copy(abuf, out_ref.at[pl.ds(base, EB)])
            return carry

        lax.fori_loop(0, nblk, body, 0)

    return k(ta, tb, src1d, dst1d)


def _sc_gather_head(hab, src1d, dst1d):
    EB = 256
    G2 = EB // 128

    @functools.partial(
        pl.kernel,
        out_type=jax.ShapeDtypeStruct((EE, 64), jnp.float32),
        mesh=_mesh(),
        scratch_types=[
            pltpu.VMEM((EB,), jnp.int32),
            pltpu.VMEM((EB,), jnp.int32),
            pltpu.VMEM((G2, 128), jnp.int32),
            pltpu.VMEM((G2, 128), jnp.int32),
            pltpu.VMEM((EB, HID), jnp.float32),
            pltpu.VMEM((EB, HID), jnp.float32),
            pltpu.VMEM((EB, 64), jnp.float32),
            pltpu.SemaphoreType.DMA,
        ],
    )
    def k(tab_ref, s_ref, d_ref, out_ref,
          ids1, idd1, ids, idd, abuf, bbuf, obuf, sem):
        w = lax.axis_index("s") * SC_CORES + lax.axis_index("c")
        nblk = jnp.where(w == 31, 10, 40)

        def body(b, carry):
            base = w * 10240 + b * EB
            pltpu.sync_copy(s_ref.at[pl.ds(base, EB)], ids1)
            pltpu.sync_copy(d_ref.at[pl.ds(base, EB)], idd1)
            for g in range(G2):
                for j in range(8):
                    jl = pl.ds(j * 16, 16)
                    fl = pl.ds(g * 128 + j * 16, 16)
                    ids[g, jl] = ids1[fl]
                    idd[g, jl] = idd1[fl]
            cps = []
            for g in range(G2):
                sl = pl.ds(g * 128, 128)
                cps.append(pltpu.async_copy(tab_ref.at[ids.at[g]], abuf.at[sl], sem))
                cps.append(pltpu.async_copy(tab_ref.at[idd.at[g]], bbuf.at[sl], sem))
            for cp in cps:
                cp.wait()

            def add_row(r, c):
                for j in range(4):
                    obuf[r, pl.ds(j * 16, 16)] = (
                        abuf[r, pl.ds(j * 16, 16)]
                        + bbuf[r, pl.ds(64 + j * 16, 16)])
                return c

            lax.fori_loop(0, EB, add_row, 0)
            pltpu.sync_copy(obuf, out_ref.at[pl.ds(base, EB)])
            return carry

        lax.fori_loop(0, nblk, body, 0)

    return k(hab, src1d, dst1d)


def _sc_agg(e_fl, a2, a3, src1d, dst1d):
    EB = 128
    STRIPE = 624
    ZR = 8

    out2 = [jax.ShapeDtypeStruct((2 * NN, HID), jnp.float32)] * 2

    @functools.partial(
        pl.kernel,
        out_type=out2,
        mesh=_mesh(),
        scratch_types=[
            pltpu.VMEM((EB,), jnp.int32),
            pltpu.VMEM((EB,), jnp.int32),
            pltpu.VMEM((1, 128), jnp.int32),
            pltpu.VMEM((1, 128), jnp.int32),
            pltpu.VMEM((EB, 64), jnp.float32),
            pltpu.VMEM((EB, HID), jnp.float32),
            pltpu.VMEM((ZR, HID), jnp.float32),
            pltpu.VMEM_SHARED((NN, HID), jnp.float32),
            pltpu.SemaphoreType.DMA,
        ],
    )
    def k(e_ref, a2_ref, a3_ref, s_ref, d_ref, f_o, b_o,
          ids1, idd1, idg, idc, ebuf, abuf, zbuf, acc, sem):
        t = lax.axis_index("s")
        core = lax.axis_index("c")
        is_last = t == SC_TILES - 1

        def zrow(r, c):
            for j in range(8):
                zbuf[r, pl.ds(j * 16, 16)] = jnp.zeros((16,), jnp.float32)
            return c

        lax.fori_loop(0, ZR, zrow, 0)

        def zero_acc():
            nq = jnp.where(is_last, (STRIPE + 16) // ZR, STRIPE // ZR)

            def zq(q, c):
                pltpu.sync_copy(zbuf, acc.at[pl.ds(t * STRIPE + q * ZR, ZR)])
                return c

            lax.fori_loop(0, nq, zq, 0)

        def run_pass(h, fwd, tab_ref, out_ref):
            nblk = jnp.where(is_last, 100, 160)

            def body(b, carry):
                base = t * 20480 + b * EB
                pltpu.sync_copy(s_ref.at[pl.ds(base, EB)], ids1)
                pltpu.sync_copy(d_ref.at[pl.ds(base, EB)], idd1)
                gsrc, csrc = (ids1, idd1) if fwd else (idd1, ids1)
                for j in range(8):
                    jl = pl.ds(j * 16, 16)
                    idg[0, jl] = gsrc[jl]
                    idc[0, jl] = csrc[jl]
                cp = pltpu.async_copy(tab_ref.at[idg.at[0]], abuf, sem)
                pltpu.sync_copy(e_ref.at[pl.ds(h * EE + base, EB)], ebuf)
                cp.wait()

                def erow(r, c):
                    for j in range(4):
                        jl = pl.ds(j * 16, 16)
                        sg = 1.0 / (1.0 + jnp.exp(-ebuf[r, jl]))
                        av = abuf[r, pl.ds(h * 64 + j * 16, 16)]
                        abuf[r, jl] = sg * av
                        abuf[r, pl.ds(64 + j * 16, 16)] = sg
                    return c

                lax.fori_loop(0, EB, erow, 0)
                pltpu.sync_copy(abuf, acc.at[idc.at[0]], add=True)
                return carry

            lax.fori_loop(0, nblk, body, 0)
            plsc.subcore_barrier()
            pltpu.sync_copy(acc.at[pl.ds(t * STRIPE, STRIPE)],
                            out_ref.at[pl.ds(h * NN + t * STRIPE, STRIPE)])

            @pl.when(is_last)
            def _tail():
                pltpu.sync_copy(
                    acc.at[pl.ds(SC_TILES * STRIPE, NN - SC_TILES * STRIPE)],
                    out_ref.at[pl.ds(h * NN + SC_TILES * STRIPE,
                                     NN - SC_TILES * STRIPE)])

        zero_acc()
        plsc.subcore_barrier()
        for h in range(2):
            @pl.when(core == 0)
            def _fwd():
                run_pass(h, True, a2_ref, f_o)

            @pl.when(core == 1)
            def _bwd():
                run_pass(h, False, a3_ref, b_o)

            if h == 0:
                plsc.subcore_barrier()
                zero_acc()
                plsc.subcore_barrier()

    return k(e_fl, a2, a3, src1d, dst1d)


def _tc_mlp(x, w1, b1, w2, b2, block_rows):
    rows, din = x.shape
    dout = w2.shape[1]

    def body(x_ref, w1_ref, b1_ref, w2_ref, b2_ref, o_ref):
        hval = jnp.maximum(x_ref[...] @ w1_ref[...] + b1_ref[...], 0.0)
        o_ref[...] = hval @ w2_ref[...] + b2_ref[...]

    return pl.pallas_call(
        body,
        grid=(rows // block_rows,),
        in_specs=[
            pl.BlockSpec((block_rows, din), lambda i: (i, 0)),
            pl.BlockSpec(w1.shape, lambda i: (0, 0)),
            pl.BlockSpec((1, b1.shape[-1]), lambda i: (0, 0)),
            pl.BlockSpec(w2.shape, lambda i: (0, 0)),
            pl.BlockSpec((1, dout), lambda i: (0, 0)),
        ],
        out_specs=pl.BlockSpec((block_rows, dout), lambda i: (i, 0)),
        out_shape=jax.ShapeDtypeStruct((rows, dout), jnp.float32),
    )(x, w1, b1[None, :], w2, b2[None, :])


def _tc_mlp_chunked_out(x, w1, b1, w2, b2, block_rows):
    rows, din = x.shape

    def body(x_ref, w1_ref, b1_ref, w2_ref, b2_ref, o_ref):
        hval = jnp.maximum(x_ref[...] @ w1_ref[...] + b1_ref[...], 0.0)
        y = hval @ w2_ref[...] + b2_ref[...]
        for c in range(NCH):
            o_ref[c] = y[:, c * CH:(c + 1) * CH]

    return pl.pallas_call(
        body,
        grid=(rows // block_rows,),
        in_specs=[
            pl.BlockSpec((block_rows, din), lambda i: (i, 0)),
            pl.BlockSpec(w1.shape, lambda i: (0, 0)),
            pl.BlockSpec((1, b1.shape[-1]), lambda i: (0, 0)),
            pl.BlockSpec(w2.shape, lambda i: (0, 0)),
            pl.BlockSpec((1, HID), lambda i: (0, 0)),
        ],
        out_specs=pl.BlockSpec((NCH, block_rows, CH), lambda i: (0, i, 0)),
        out_shape=jax.ShapeDtypeStruct((NCH, rows, CH), jnp.float32),
    )(x, w1, b1[None, :], w2, b2[None, :])


def _tc_node_mm(h, wcat, bcat):
    block = 2000

    def body(h_ref, w_ref, b_ref, a1_ref, a2_ref, a3_ref, b1_ref, b2_ref):
        hw = h_ref[...] @ w_ref[...] + b_ref[...]
        a1_ref[...] = hw[:, 0:128]
        a2_ref[...] = hw[:, 128:256]
        a3_ref[...] = hw[:, 256:384]
        b1_ref[...] = hw[:, 384:512]
        b2_ref[...] = hw[:, 512:640]

    ospec = pl.BlockSpec((block, HID), lambda i: (i, 0))
    oshape = jax.ShapeDtypeStruct((NN, HID), jnp.float32)
    return pl.pallas_call(
        body,
        grid=(NN // block,),
        in_specs=[
            pl.BlockSpec((block, HID), lambda i: (i, 0)),
            pl.BlockSpec((HID, 5 * HID), lambda i: (0, 0)),
            pl.BlockSpec((1, 5 * HID), lambda i: (0, 0)),
        ],
        out_specs=[ospec] * 5,
        out_shape=[oshape] * 5,
    )(h, wcat, bcat[None, :])


def _tc_chunked_matmul(e_st, w, b):
    dout = w.shape[1]

    def body(e_ref, w_ref, b_ref, o_ref):
        x = jnp.concatenate([e_ref[c] for c in range(NCH)], axis=-1)
        o_ref[...] = x @ w_ref[...] + b_ref[...]

    return pl.pallas_call(
        body,
        grid=(EE // EBT,),
        in_specs=[
            pl.BlockSpec((NCH, EBT, CH), lambda i: (0, i, 0)),
            pl.BlockSpec((HID, dout), lambda i: (0, 0)),
            pl.BlockSpec((1, dout), lambda i: (0, 0)),
        ],
        out_specs=pl.BlockSpec((EBT, dout), lambda i: (i, 0)),
        out_shape=jax.ShapeDtypeStruct((EE, dout), jnp.float32),
    )(e_st, w, b[None, :])


def _layer_norm_rows(v, g, b):
    mu = jnp.mean(v, axis=-1, keepdims=True)
    var = jnp.mean((v - mu) ** 2, axis=-1, keepdims=True)
    return (v - mu) * jax.lax.rsqrt(var + 1e-5) * g + b


def _tc_edge_update(gsum, b3e, e_st, ln_g, ln_b):
    def body(g_ref, b3_ref, e_ref, lng_ref, lnb_ref, o_ref):
        e_in = jnp.concatenate([e_ref[c] for c in range(NCH)], axis=-1)
        e_hat = _layer_norm_rows(g_ref[...] + b3_ref[...],
                                 lng_ref[...], lnb_ref[...])
        e_new = e_in + jnp.maximum(e_hat, 0.0)
        for c in range(NCH):
            o_ref[c] = e_new[:, c * CH:(c + 1) * CH]

    return pl.pallas_call(
        body,
        grid=(EE // EBT,),
        in_specs=[
            pl.BlockSpec((EBT, HID), lambda i: (i, 0)),
            pl.BlockSpec((EBT, HID), lambda i: (i, 0)),
            pl.BlockSpec((NCH, EBT, CH), lambda i: (0, i, 0)),
            pl.BlockSpec((1, HID), lambda i: (0, 0)),
            pl.BlockSpec((1, HID), lambda i: (0, 0)),
        ],
        out_specs=pl.BlockSpec((NCH, EBT, CH), lambda i: (0, i, 0)),
        out_shape=jax.ShapeDtypeStruct((NCH, EE, CH), jnp.float32),
    )(gsum, b3e, e_st, ln_g[None, :], ln_b[None, :])


def _tc_node_update(h_in, a1h, f_st, b_st, ln_g, ln_b):
    block = 2000

    def body(h_ref, a1_ref, f_ref, b_ref, lng_ref, lnb_ref, o_ref):
        nf = jnp.concatenate([f_ref[0][:, 0:64], f_ref[1][:, 0:64]], axis=-1)
        df = jnp.concatenate([f_ref[0][:, 64:128], f_ref[1][:, 64:128]], axis=-1)
        nb = jnp.concatenate([b_ref[0][:, 0:64], b_ref[1][:, 0:64]], axis=-1)
        db = jnp.concatenate([b_ref[0][:, 64:128], b_ref[1][:, 64:128]], axis=-1)
        h_hat = a1_ref[...] + nf / (df + 1e-6) + nb / (db + 1e-6)
        h_hat = jnp.maximum(
            _layer_norm_rows(h_hat, lng_ref[...], lnb_ref[...]), 0.0)
        o_ref[...] = h_ref[...] + h_hat

    st = pl.BlockSpec((2, block, HID), lambda i: (0, i, 0))
    return pl.pallas_call(
        body,
        grid=(NN // block,),
        in_specs=[
            pl.BlockSpec((block, HID), lambda i: (i, 0)),
            pl.BlockSpec((block, HID), lambda i: (i, 0)),
            st, st,
            pl.BlockSpec((1, HID), lambda i: (0, 0)),
            pl.BlockSpec((1, HID), lambda i: (0, 0)),
        ],
        out_specs=pl.BlockSpec((block, HID), lambda i: (i, 0)),
        out_shape=jax.ShapeDtypeStruct((NN, HID), jnp.float32),
    )(h_in, a1h, f_st, b_st, ln_g[None, :], ln_b[None, :])


def _tc_pred_node(h, w1ab):
    block = 2000

    def body(h_ref, w_ref, o_ref):
        o_ref[...] = h_ref[...] @ w_ref[...]

    return pl.pallas_call(
        body,
        grid=(NN // block,),
        in_specs=[
            pl.BlockSpec((block, HID), lambda i: (i, 0)),
            pl.BlockSpec((HID, HID), lambda i: (0, 0)),
        ],
        out_specs=pl.BlockSpec((block, HID), lambda i: (i, 0)),
        out_shape=jax.ShapeDtypeStruct((NN, HID), jnp.float32),
    )(h, w1ab)


def _tc_score_fin(gz, ec, w2, b2):
    def body(gz_ref, ec_ref, w2_ref, b2_ref, o_ref):
        z = jnp.maximum(gz_ref[...] + ec_ref[...], 0.0)
        o_ref[...] = jnp.sum(z * w2_ref[...], axis=-1, keepdims=True) + b2_ref[...]

    return pl.pallas_call(
        body,
        grid=(EE // EBT,),
        in_specs=[
            pl.BlockSpec((EBT, 64), lambda i: (i, 0)),
            pl.BlockSpec((EBT, 64), lambda i: (i, 0)),
            pl.BlockSpec((1, 64), lambda i: (0, 0)),
            pl.BlockSpec((1, 1), lambda i: (0, 0)),
        ],
        out_specs=pl.BlockSpec((EBT, 1), lambda i: (i, 0)),
        out_shape=jax.ShapeDtypeStruct((EE, 1), jnp.float32),
    )(gz, ec, w2[None, :], b2[None, None, 0])


def kernel(x, e, edge_index, params):
    p = params
    src = edge_index[0]
    dst = edge_index[1]

    h = _tc_mlp(x, p['ne_W1'], p['ne_b1'], p['ne_W2'], p['ne_b2'], 2000)
    e_st = _tc_mlp_chunked_out(e, p['ee_W1'], p['ee_b1'], p['ee_W2'],
                               p['ee_b2'], EBT)

    for i in range(LAYERS):
        wcat = jnp.concatenate(
            [p['A1_W'][i], p['A2_W'][i], p['A3_W'][i],
             p['B1_W'][i], p['B2_W'][i]], axis=1)
        bcat = jnp.concatenate(
            [p['A1_b'][i], p['A2_b'][i], p['A3_b'][i],
             p['B1_b'][i], p['B2_b'][i]], axis=0)
        a1h, a2h, a3h, b1h, b2h = _tc_node_mm(h, wcat, bcat)
        b3e = _tc_chunked_matmul(e_st, p['B3_W'][i], p['B3_b'][i])
        gsum = _sc_gather2add(b1h, b2h, src, dst, HID)
        e_st = _tc_edge_update(gsum, b3e, e_st, p['ln_e_g'][i], p['ln_e_b'][i])
        e_fl = e_st.reshape(NCH * EE, CH)
        f_fl, b_fl = _sc_agg(e_fl, a2h, a3h, src, dst)
        h = _tc_node_update(h, a1h, f_fl.reshape(2, NN, HID),
                            b_fl.reshape(2, NN, HID),
                            p['ln_h_g'][i], p['ln_h_b'][i])

    w1 = p['pred_W1']
    hab = _tc_pred_node(h, w1[0:2 * HID].reshape(2, HID, 64)
                        .transpose(1, 0, 2).reshape(HID, HID))
    ec = _tc_chunked_matmul(e_st, w1[2 * HID:3 * HID], p['pred_b1'])
    gz = _sc_gather_head(hab, src, dst)
    return _tc_score_fin(gz, ec, p['pred_W2'][:, 0], p['pred_b2'])


# speedup vs baseline: 3.2431x; 1.1452x over previous
import functools

import jax
import jax.numpy as jnp
from jax import lax
from jax.experimental import pallas as pl
from jax.experimental.pallas import tpu as pltpu
from jax.experimental.pallas import tpu_sc as plsc

NN = 10000
EE = 320000
HID = 128
NCH = 2
CH = 64
LAYERS = 4
SC_CORES = 2
SC_TILES = 16

EBT = 2000


def _mesh():
    return plsc.VectorSubcoreMesh(
        core_axis_name="c", subcore_axis_name="s",
        num_cores=SC_CORES, num_subcores=SC_TILES)


def _sc_gather2add(ta, tb, src1d, dst1d, D):
    EB = 256
    G2 = EB // 128

    @functools.partial(
        pl.kernel,
        out_type=jax.ShapeDtypeStruct((EE, D), jnp.float32),
        mesh=_mesh(),
        scratch_types=[
            pltpu.VMEM((EB,), jnp.int32),
            pltpu.VMEM((EB,), jnp.int32),
            pltpu.VMEM((G2, 128), jnp.int32),
            pltpu.VMEM((G2, 128), jnp.int32),
            pltpu.VMEM((EB, D), jnp.float32),
            pltpu.VMEM((EB, D), jnp.float32),
            pltpu.SemaphoreType.DMA,
        ],
    )
    def k(ta_ref, tb_ref, s_ref, d_ref, out_ref,
          ids1, idd1, ids, idd, abuf, bbuf, sem):
        w = lax.axis_index("s") * SC_CORES + lax.axis_index("c")
        nblk = jnp.where(w == 31, 10, 40)

        def body(b, carry):
            base = w * 10240 + b * EB
            pltpu.sync_copy(s_ref.at[pl.ds(base, EB)], ids1)
            pltpu.sync_copy(d_ref.at[pl.ds(base, EB)], idd1)
            for g in range(G2):
                for j in range(8):
                    jl = pl.ds(j * 16, 16)
                    fl = pl.ds(g * 128 + j * 16, 16)
                    ids[g, jl] = ids1[fl]
                    idd[g, jl] = idd1[fl]
            cps = []
            for g in range(G2):
                sl = pl.ds(g * 128, 128)
                cps.append(pltpu.async_copy(ta_ref.at[ids.at[g]], abuf.at[sl], sem))
                cps.append(pltpu.async_copy(tb_ref.at[idd.at[g]], bbuf.at[sl], sem))
            for cp in cps:
                cp.wait()

            def add_row(r, c):
                for j in range(D // 16):
                    jl = pl.ds(j * 16, 16)
                    abuf[r, jl] = abuf[r, jl] + bbuf[r, jl]
                return c

            lax.fori_loop(0, EB, add_row, 0)
            pltpu.sync_copy(abuf, out_ref.at[pl.ds(base, EB)])
            return carry

        lax.fori_loop(0, nblk, body, 0)

    return k(ta, tb, src1d, dst1d)


def _sc_gather_head(hab, src1d, dst1d):
    EB = 256
    G2 = EB // 128

    @functools.partial(
        pl.kernel,
        out_type=jax.ShapeDtypeStruct((EE, 64), jnp.float32),
        mesh=_mesh(),
        scratch_types=[
            pltpu.VMEM((EB,), jnp.int32),
            pltpu.VMEM((EB,), jnp.int32),
            pltpu.VMEM((G2, 128), jnp.int32),
            pltpu.VMEM((G2, 128), jnp.int32),
            pltpu.VMEM((EB, HID), jnp.float32),
            pltpu.VMEM((EB, HID), jnp.float32),
            pltpu.VMEM((EB, 64), jnp.float32),
            pltpu.SemaphoreType.DMA,
        ],
    )
    def k(tab_ref, s_ref, d_ref, out_ref,
          ids1, idd1, ids, idd, abuf, bbuf, obuf, sem):
        w = lax.axis_index("s") * SC_CORES + lax.axis_index("c")
        nblk = jnp.where(w == 31, 10, 40)

        def body(b, carry):
            base = w * 10240 + b * EB
            pltpu.sync_copy(s_ref.at[pl.ds(base, EB)], ids1)
            pltpu.sync_copy(d_ref.at[pl.ds(base, EB)], idd1)
            for g in range(G2):
                for j in range(8):
                    jl = pl.ds(j * 16, 16)
                    fl = pl.ds(g * 128 + j * 16, 16)
                    ids[g, jl] = ids1[fl]
                    idd[g, jl] = idd1[fl]
            cps = []
            for g in range(G2):
                sl = pl.ds(g * 128, 128)
                cps.append(pltpu.async_copy(tab_ref.at[ids.at[g]], abuf.at[sl], sem))
                cps.append(pltpu.async_copy(tab_ref.at[idd.at[g]], bbuf.at[sl], sem))
            for cp in cps:
                cp.wait()

            def add_row(r, c):
                for j in range(4):
                    obuf[r, pl.ds(j * 16, 16)] = (
                        abuf[r, pl.ds(j * 16, 16)]
                        + bbuf[r, pl.ds(64 + j * 16, 16)])
                return c

            lax.fori_loop(0, EB, add_row, 0)
            pltpu.sync_copy(obuf, out_ref.at[pl.ds(base, EB)])
            return carry

        lax.fori_loop(0, nblk, body, 0)

    return k(hab, src1d, dst1d)


def _sc_agg(e_fl, a2, a3, src1d, dst1d):
    EB = 64
    SB = 8 * EB
    STRIPE = 624
    ZR = 8

    out2 = [jax.ShapeDtypeStruct((2 * NN, HID), jnp.float32)] * 2

    @functools.partial(
        pl.kernel,
        out_type=out2,
        mesh=_mesh(),
        scratch_types=[
            pltpu.VMEM((SB,), jnp.int32),
            pltpu.VMEM((SB,), jnp.int32),
            pltpu.VMEM((2, 1, EB), jnp.int32),
            pltpu.VMEM((2, 1, EB), jnp.int32),
            pltpu.VMEM((2, EB, 64), jnp.float32),
            pltpu.VMEM((2, EB, HID), jnp.float32),
            pltpu.VMEM((ZR, HID), jnp.float32),
            pltpu.VMEM_SHARED((NN, HID), jnp.float32),
            pltpu.SemaphoreType.DMA,
            pltpu.SemaphoreType.DMA,
            pltpu.SemaphoreType.DMA,
            pltpu.SemaphoreType.DMA,
            pltpu.SemaphoreType.DMA,
            pltpu.SemaphoreType.DMA,
        ],
    )
    def k(e_ref, a2_ref, a3_ref, s_ref, d_ref, f_o, b_o,
          ids1, idd1, idg, idc, ebuf, abuf, zbuf, acc,
          sg0, sg1, se0, se1, ss0, ss1, *_):
        t = lax.axis_index("s")
        core = lax.axis_index("c")
        is_last = t == SC_TILES - 1
        sgs = (sg0, sg1)
        ses = (se0, se1)
        sss = (ss0, ss1)

        def zrow(r, c):
            for j in range(8):
                zbuf[r, pl.ds(j * 16, 16)] = jnp.zeros((16,), jnp.float32)
            return c

        lax.fori_loop(0, ZR, zrow, 0)

        def zero_acc():
            nq = jnp.where(is_last, (STRIPE + 16) // ZR, STRIPE // ZR)

            def zq(q, c):
                pltpu.sync_copy(zbuf, acc.at[pl.ds(t * STRIPE + q * ZR, ZR)])
                return c

            lax.fori_loop(0, nq, zq, 0)

        def run_pass(h, fwd, tab_ref, out_ref):
            nblk = jnp.where(is_last, 200, 320)
            tbase = t * 20480

            def stage_super(b0):
                pltpu.sync_copy(s_ref.at[pl.ds(tbase + b0 * EB, SB)], ids1)
                pltpu.sync_copy(d_ref.at[pl.ds(tbase + b0 * EB, SB)], idd1)

            def transform(b, slot):
                g = b % 8
                gsrc, csrc = (ids1, idd1) if fwd else (idd1, ids1)
                for j in range(EB // 16):
                    jl = pl.ds(j * 16, 16)
                    fl = pl.ds(g * EB + j * 16, 16)
                    idg[slot, 0, jl] = gsrc[fl]
                    idc[slot, 0, jl] = csrc[fl]

            def issue(b, slot):
                pltpu.async_copy(tab_ref.at[idg.at[slot, 0]],
                                 abuf.at[slot], sgs[slot])
                pltpu.async_copy(e_ref.at[pl.ds(h * EE + tbase + b * EB, EB)],
                                 ebuf.at[slot], ses[slot])

            def wait_in(slot):
                pltpu.make_async_copy(tab_ref.at[idg.at[slot, 0]],
                                      abuf.at[slot], sgs[slot]).wait()
                pltpu.make_async_copy(e_ref.at[pl.ds(0, EB)],
                                      ebuf.at[slot], ses[slot]).wait()

            def wait_scatter(slot):
                pltpu.make_async_copy(abuf.at[slot],
                                      acc.at[idc.at[slot, 0]],
                                      sss[slot]).wait()

            stage_super(0)
            transform(0, 0)
            issue(0, 0)

            def body(b, carry):
                for par in range(2):

                    @pl.when((b * 2 + par) < nblk)
                    def _sub():
                        bb = b * 2 + par
                        wait_in(par)

                        def erow(r, c):
                            for j in range(4):
                                jl = pl.ds(j * 16, 16)
                                sg = 1.0 / (1.0 + jnp.exp(-ebuf[par, r, jl]))
                                av = abuf[par, r, pl.ds(h * 64 + j * 16, 16)]
                                abuf[par, r, jl] = sg * av
                                abuf[par, r, pl.ds(64 + j * 16, 16)] = sg
                            return c

                        lax.fori_loop(0, EB, erow, 0)
                        pltpu.async_copy(abuf.at[par],
                                         acc.at[idc.at[par, 0]],
                                         sss[par], add=True)

                        nxt = 1 - par

                        @pl.when(bb + 1 < nblk)
                        def _prefetch():
                            @pl.when(bb + 1 >= 2)
                            def _w1():
                                wait_scatter(nxt)

                            transform(bb + 1, nxt)
                            issue(bb + 1, nxt)

                        @pl.when(jnp.logical_and((bb + 2) % 8 == 0,
                                                 bb + 2 < nblk))
                        def _stage():
                            stage_super(bb + 2)

                return carry

            lax.fori_loop(0, 160, body, 0)
            wait_scatter(0)
            wait_scatter(1)
            plsc.subcore_barrier()
            pltpu.sync_copy(acc.at[pl.ds(t * STRIPE, STRIPE)],
                            out_ref.at[pl.ds(h * NN + t * STRIPE, STRIPE)])

            @pl.when(is_last)
            def _tail():
                pltpu.sync_copy(
                    acc.at[pl.ds(SC_TILES * STRIPE, NN - SC_TILES * STRIPE)],
                    out_ref.at[pl.ds(h * NN + SC_TILES * STRIPE,
                                     NN - SC_TILES * STRIPE)])

        zero_acc()
        plsc.subcore_barrier()
        for h in range(2):
            @pl.when(core == 0)
            def _fwd():
                run_pass(h, True, a2_ref, f_o)

            @pl.when(core == 1)
            def _bwd():
                run_pass(h, False, a3_ref, b_o)

            if h == 0:
                plsc.subcore_barrier()
                zero_acc()
                plsc.subcore_barrier()

    return k(e_fl, a2, a3, src1d, dst1d)


def _tc_mlp(x, w1, b1, w2, b2, block_rows):
    rows, din = x.shape
    dout = w2.shape[1]

    def body(x_ref, w1_ref, b1_ref, w2_ref, b2_ref, o_ref):
        hval = jnp.maximum(x_ref[...] @ w1_ref[...] + b1_ref[...], 0.0)
        o_ref[...] = hval @ w2_ref[...] + b2_ref[...]

    return pl.pallas_call(
        body,
        grid=(rows // block_rows,),
        in_specs=[
            pl.BlockSpec((block_rows, din), lambda i: (i, 0)),
            pl.BlockSpec(w1.shape, lambda i: (0, 0)),
            pl.BlockSpec((1, b1.shape[-1]), lambda i: (0, 0)),
            pl.BlockSpec(w2.shape, lambda i: (0, 0)),
            pl.BlockSpec((1, dout), lambda i: (0, 0)),
        ],
        out_specs=pl.BlockSpec((block_rows, dout), lambda i: (i, 0)),
        out_shape=jax.ShapeDtypeStruct((rows, dout), jnp.float32),
    )(x, w1, b1[None, :], w2, b2[None, :])


def _tc_mlp_chunked_out(x, w1, b1, w2, b2, block_rows):
    rows, din = x.shape

    def body(x_ref, w1_ref, b1_ref, w2_ref, b2_ref, o_ref):
        hval = jnp.maximum(x_ref[...] @ w1_ref[...] + b1_ref[...], 0.0)
        y = hval @ w2_ref[...] + b2_ref[...]
        for c in range(NCH):
            o_ref[c] = y[:, c * CH:(c + 1) * CH]

    return pl.pallas_call(
        body,
        grid=(rows // block_rows,),
        in_specs=[
            pl.BlockSpec((block_rows, din), lambda i: (i, 0)),
            pl.BlockSpec(w1.shape, lambda i: (0, 0)),
            pl.BlockSpec((1, b1.shape[-1]), lambda i: (0, 0)),
            pl.BlockSpec(w2.shape, lambda i: (0, 0)),
            pl.BlockSpec((1, HID), lambda i: (0, 0)),
        ],
        out_specs=pl.BlockSpec((NCH, block_rows, CH), lambda i: (0, i, 0)),
        out_shape=jax.ShapeDtypeStruct((NCH, rows, CH), jnp.float32),
    )(x, w1, b1[None, :], w2, b2[None, :])


def _tc_node_mm(h, wcat, bcat):
    block = 2000

    def body(h_ref, w_ref, b_ref, a1_ref, a2_ref, a3_ref, b1_ref, b2_ref):
        hw = h_ref[...] @ w_ref[...] + b_ref[...]
        a1_ref[...] = hw[:, 0:128]
        a2_ref[...] = hw[:, 128:256]
        a3_ref[...] = hw[:, 256:384]
        b1_ref[...] = hw[:, 384:512]
        b2_ref[...] = hw[:, 512:640]

    ospec = pl.BlockSpec((block, HID), lambda i: (i, 0))
    oshape = jax.ShapeDtypeStruct((NN, HID), jnp.float32)
    return pl.pallas_call(
        body,
        grid=(NN // block,),
        in_specs=[
            pl.BlockSpec((block, HID), lambda i: (i, 0)),
            pl.BlockSpec((HID, 5 * HID), lambda i: (0, 0)),
            pl.BlockSpec((1, 5 * HID), lambda i: (0, 0)),
        ],
        out_specs=[ospec] * 5,
        out_shape=[oshape] * 5,
    )(h, wcat, bcat[None, :])


def _tc_chunked_matmul(e_st, w, b):
    dout = w.shape[1]

    def body(e_ref, w_ref, b_ref, o_ref):
        x = jnp.concatenate([e_ref[c] for c in range(NCH)], axis=-1)
        o_ref[...] = x @ w_ref[...] + b_ref[...]

    return pl.pallas_call(
        body,
        grid=(EE // EBT,),
        in_specs=[
            pl.BlockSpec((NCH, EBT, CH), lambda i: (0, i, 0)),
            pl.BlockSpec((HID, dout), lambda i: (0, 0)),
            pl.BlockSpec((1, dout), lambda i: (0, 0)),
        ],
        out_specs=pl.BlockSpec((EBT, dout), lambda i: (i, 0)),
        out_shape=jax.ShapeDtypeStruct((EE, dout), jnp.float32),
    )(e_st, w, b[None, :])


def _layer_norm_rows(v, g, b):
    mu = jnp.mean(v, axis=-1, keepdims=True)
    var = jnp.mean((v - mu) ** 2, axis=-1, keepdims=True)
    return (v - mu) * jax.lax.rsqrt(var + 1e-5) * g + b


def _tc_edge_update(gsum, b3e, e_st, ln_g, ln_b):
    def body(g_ref, b3_ref, e_ref, lng_ref, lnb_ref, o_ref):
        e_in = jnp.concatenate([e_ref[c] for c in range(NCH)], axis=-1)
        e_hat = _layer_norm_rows(g_ref[...] + b3_ref[...],
                                 lng_ref[...], lnb_ref[...])
        e_new = e_in + jnp.maximum(e_hat, 0.0)
        for c in range(NCH):
            o_ref[c] = e_new[:, c * CH:(c + 1) * CH]

    return pl.pallas_call(
        body,
        grid=(EE // EBT,),
        in_specs=[
            pl.BlockSpec((EBT, HID), lambda i: (i, 0)),
            pl.BlockSpec((EBT, HID), lambda i: (i, 0)),
            pl.BlockSpec((NCH, EBT, CH), lambda i: (0, i, 0)),
            pl.BlockSpec((1, HID), lambda i: (0, 0)),
            pl.BlockSpec((1, HID), lambda i: (0, 0)),
        ],
        out_specs=pl.BlockSpec((NCH, EBT, CH), lambda i: (0, i, 0)),
        out_shape=jax.ShapeDtypeStruct((NCH, EE, CH), jnp.float32),
    )(gsum, b3e, e_st, ln_g[None, :], ln_b[None, :])


def _tc_node_update(h_in, a1h, f_st, b_st, ln_g, ln_b):
    block = 2000

    def body(h_ref, a1_ref, f_ref, b_ref, lng_ref, lnb_ref, o_ref):
        nf = jnp.concatenate([f_ref[0][:, 0:64], f_ref[1][:, 0:64]], axis=-1)
        df = jnp.concatenate([f_ref[0][:, 64:128], f_ref[1][:, 64:128]], axis=-1)
        nb = jnp.concatenate([b_ref[0][:, 0:64], b_ref[1][:, 0:64]], axis=-1)
        db = jnp.concatenate([b_ref[0][:, 64:128], b_ref[1][:, 64:128]], axis=-1)
        h_hat = a1_ref[...] + nf / (df + 1e-6) + nb / (db + 1e-6)
        h_hat = jnp.maximum(
            _layer_norm_rows(h_hat, lng_ref[...], lnb_ref[...]), 0.0)
        o_ref[...] = h_ref[...] + h_hat

    st = pl.BlockSpec((2, block, HID), lambda i: (0, i, 0))
    return pl.pallas_call(
        body,
        grid=(NN // block,),
        in_specs=[
            pl.BlockSpec((block, HID), lambda i: (i, 0)),
            pl.BlockSpec((block, HID), lambda i: (i, 0)),
            st, st,
            pl.BlockSpec((1, HID), lambda i: (0, 0)),
            pl.BlockSpec((1, HID), lambda i: (0, 0)),
        ],
        out_specs=pl.BlockSpec((block, HID), lambda i: (i, 0)),
        out_shape=jax.ShapeDtypeStruct((NN, HID), jnp.float32),
    )(h_in, a1h, f_st, b_st, ln_g[None, :], ln_b[None, :])


def _tc_pred_node(h, w1ab):
    block = 2000

    def body(h_ref, w_ref, o_ref):
        o_ref[...] = h_ref[...] @ w_ref[...]

    return pl.pallas_call(
        body,
        grid=(NN // block,),
        in_specs=[
            pl.BlockSpec((block, HID), lambda i: (i, 0)),
            pl.BlockSpec((HID, HID), lambda i: (0, 0)),
        ],
        out_specs=pl.BlockSpec((block, HID), lambda i: (i, 0)),
        out_shape=jax.ShapeDtypeStruct((NN, HID), jnp.float32),
    )(h, w1ab)


def _tc_score_fin(gz, ec, w2, b2):
    def body(gz_ref, ec_ref, w2_ref, b2_ref, o_ref):
        z = jnp.maximum(gz_ref[...] + ec_ref[...], 0.0)
        o_ref[...] = jnp.sum(z * w2_ref[...], axis=-1, keepdims=True) + b2_ref[...]

    return pl.pallas_call(
        body,
        grid=(EE // EBT,),
        in_specs=[
            pl.BlockSpec((EBT, 64), lambda i: (i, 0)),
            pl.BlockSpec((EBT, 64), lambda i: (i, 0)),
            pl.BlockSpec((1, 64), lambda i: (0, 0)),
            pl.BlockSpec((1, 1), lambda i: (0, 0)),
        ],
        out_specs=pl.BlockSpec((EBT, 1), lambda i: (i, 0)),
        out_shape=jax.ShapeDtypeStruct((EE, 1), jnp.float32),
    )(gz, ec, w2[None, :], b2[None, None, 0])


def kernel(x, e, edge_index, params):
    p = params
    src = edge_index[0]
    dst = edge_index[1]

    h = _tc_mlp(x, p['ne_W1'], p['ne_b1'], p['ne_W2'], p['ne_b2'], 2000)
    e_st = _tc_mlp_chunked_out(e, p['ee_W1'], p['ee_b1'], p['ee_W2'],
                               p['ee_b2'], EBT)

    for i in range(LAYERS):
        wcat = jnp.concatenate(
            [p['A1_W'][i], p['A2_W'][i], p['A3_W'][i],
             p['B1_W'][i], p['B2_W'][i]], axis=1)
        bcat = jnp.concatenate(
            [p['A1_b'][i], p['A2_b'][i], p['A3_b'][i],
             p['B1_b'][i], p['B2_b'][i]], axis=0)
        a1h, a2h, a3h, b1h, b2h = _tc_node_mm(h, wcat, bcat)
        b3e = _tc_chunked_matmul(e_st, p['B3_W'][i], p['B3_b'][i])
        gsum = _sc_gather2add(b1h, b2h, src, dst, HID)
        e_st = _tc_edge_update(gsum, b3e, e_st, p['ln_e_g'][i], p['ln_e_b'][i])
        e_fl = e_st.reshape(NCH * EE, CH)
        f_fl, b_fl = _sc_agg(e_fl, a2h, a3h, src, dst)
        h = _tc_node_update(h, a1h, f_fl.reshape(2, NN, HID),
                            b_fl.reshape(2, NN, HID),
                            p['ln_h_g'][i], p['ln_h_b'][i])

    w1 = p['pred_W1']
    hab = _tc_pred_node(h, w1[0:2 * HID].reshape(2, HID, 64)
                        .transpose(1, 0, 2).reshape(HID, HID))
    ec = _tc_chunked_matmul(e_st, w1[2 * HID:3 * HID], p['pred_b1'])
    gz = _sc_gather_head(hab, src, dst)
    return _tc_score_fin(gz, ec, p['pred_W2'][:, 0], p['pred_b2'])


# speedup vs baseline: 3.3343x; 1.0281x over previous
import functools

import jax
import jax.numpy as jnp
from jax import lax
from jax.experimental import pallas as pl
from jax.experimental.pallas import tpu as pltpu
from jax.experimental.pallas import tpu_sc as plsc

NN = 10000
EE = 320000
HID = 128
NCH = 2
CH = 64
LAYERS = 4
SC_CORES = 2
SC_TILES = 16

EBT = 2000


def _mesh():
    return plsc.VectorSubcoreMesh(
        core_axis_name="c", subcore_axis_name="s",
        num_cores=SC_CORES, num_subcores=SC_TILES)


def _sc_gather_kernel(ta, tb, src1d, dst1d, head):
    EB = 128
    SB = 8 * EB
    DO = 64 if head else HID

    scratch = [
        pltpu.VMEM((SB,), jnp.int32),
        pltpu.VMEM((SB,), jnp.int32),
        pltpu.VMEM((2, 1, EB), jnp.int32),
        pltpu.VMEM((2, 1, EB), jnp.int32),
        pltpu.VMEM((2, EB, HID), jnp.float32),
        pltpu.VMEM((2, EB, HID), jnp.float32),
        pltpu.SemaphoreType.DMA,
        pltpu.SemaphoreType.DMA,
        pltpu.SemaphoreType.DMA,
        pltpu.SemaphoreType.DMA,
        pltpu.SemaphoreType.DMA,
        pltpu.SemaphoreType.DMA,
    ]
    if head:
        scratch.insert(6, pltpu.VMEM((2, EB, 64), jnp.float32))

    @functools.partial(
        pl.kernel,
        out_type=jax.ShapeDtypeStruct((EE, DO), jnp.float32),
        mesh=_mesh(),
        scratch_types=scratch,
    )
    def k(ta_ref, tb_ref, s_ref, d_ref, out_ref, *scr):
        if head:
            ids1, idd1, iga, igb, abuf, bbuf, obuf = scr[:7]
            sems = scr[7:]
        else:
            ids1, idd1, iga, igb, abuf, bbuf = scr[:6]
            obuf = abuf
            sems = scr[6:]
        ga = sems[0:2]
        gb = sems[2:4]
        wr = sems[4:6]
        w = lax.axis_index("s") * SC_CORES + lax.axis_index("c")
        nblk = jnp.where(w == 31, 20, 80)
        wbase = w * 10240

        def stage_super(b0):
            pltpu.sync_copy(s_ref.at[pl.ds(wbase + b0 * EB, SB)], ids1)
            pltpu.sync_copy(d_ref.at[pl.ds(wbase + b0 * EB, SB)], idd1)

        def transform(b, slot):
            g = b % 8
            for j in range(EB // 16):
                jl = pl.ds(j * 16, 16)
                fl = pl.ds(g * EB + j * 16, 16)
                iga[slot, 0, jl] = ids1[fl]
                igb[slot, 0, jl] = idd1[fl]

        def issue(b, slot):
            pltpu.async_copy(ta_ref.at[iga.at[slot, 0]], abuf.at[slot], ga[slot])
            pltpu.async_copy(tb_ref.at[igb.at[slot, 0]], bbuf.at[slot], gb[slot])

        def wait_in(slot):
            pltpu.make_async_copy(ta_ref.at[iga.at[slot, 0]],
                                  abuf.at[slot], ga[slot]).wait()
            pltpu.make_async_copy(tb_ref.at[igb.at[slot, 0]],
                                  bbuf.at[slot], gb[slot]).wait()

        def wait_write(slot):
            pltpu.make_async_copy(obuf.at[slot],
                                  out_ref.at[pl.ds(0, EB)], wr[slot]).wait()

        stage_super(0)
        transform(0, 0)
        issue(0, 0)

        def body(b, carry):
            for par in range(2):

                @pl.when((b * 2 + par) < nblk)
                def _sub():
                    bb = b * 2 + par
                    wait_in(par)

                    def add_row(r, c):
                        if head:
                            for j in range(4):
                                obuf[par, r, pl.ds(j * 16, 16)] = (
                                    abuf[par, r, pl.ds(j * 16, 16)]
                                    + bbuf[par, r, pl.ds(64 + j * 16, 16)])
                        else:
                            for j in range(8):
                                jl = pl.ds(j * 16, 16)
                                abuf[par, r, jl] = (abuf[par, r, jl]
                                                    + bbuf[par, r, jl])
                        return c

                    lax.fori_loop(0, EB, add_row, 0)
                    pltpu.async_copy(
                        obuf.at[par],
                        out_ref.at[pl.ds(wbase + bb * EB, EB)], wr[par])
                    nxt = 1 - par

                    @pl.when(bb + 1 < nblk)
                    def _prefetch():
                        @pl.when(bb + 1 >= 2)
                        def _w1():
                            wait_write(nxt)

                        transform(bb + 1, nxt)
                        issue(bb + 1, nxt)

                    @pl.when(jnp.logical_and((bb + 2) % 8 == 0,
                                             bb + 2 < nblk))
                    def _stage():
                        stage_super(bb + 2)

            return carry

        lax.fori_loop(0, 40, body, 0)
        wait_write(0)
        wait_write(1)

    return k(ta, tb, src1d, dst1d)


def _sc_gather2add(ta, tb, src1d, dst1d, D):
    return _sc_gather_kernel(ta, tb, src1d, dst1d, head=False)


def _sc_gather_head(hab, src1d, dst1d):
    return _sc_gather_kernel(hab, hab, src1d, dst1d, head=True)


def _sc_agg(e_fl, a2, a3, src1d, dst1d):
    EB = 64
    SB = 8 * EB
    STRIPE = 624
    ZR = 8

    out2 = [jax.ShapeDtypeStruct((2 * NN, HID), jnp.float32)] * 2

    @functools.partial(
        pl.kernel,
        out_type=out2,
        mesh=_mesh(),
        scratch_types=[
            pltpu.VMEM((SB,), jnp.int32),
            pltpu.VMEM((SB,), jnp.int32),
            pltpu.VMEM((2, 1, EB), jnp.int32),
            pltpu.VMEM((2, 1, EB), jnp.int32),
            pltpu.VMEM((2, EB, 64), jnp.float32),
            pltpu.VMEM((2, EB, HID), jnp.float32),
            pltpu.VMEM((ZR, HID), jnp.float32),
            pltpu.VMEM_SHARED((NN, HID), jnp.float32),
            pltpu.SemaphoreType.DMA,
            pltpu.SemaphoreType.DMA,
            pltpu.SemaphoreType.DMA,
            pltpu.SemaphoreType.DMA,
            pltpu.SemaphoreType.DMA,
            pltpu.SemaphoreType.DMA,
        ],
    )
    def k(e_ref, a2_ref, a3_ref, s_ref, d_ref, f_o, b_o,
          ids1, idd1, idg, idc, ebuf, abuf, zbuf, acc,
          sg0, sg1, se0, se1, ss0, ss1, *_):
        t = lax.axis_index("s")
        core = lax.axis_index("c")
        is_last = t == SC_TILES - 1
        sgs = (sg0, sg1)
        ses = (se0, se1)
        sss = (ss0, ss1)

        def zrow(r, c):
            for j in range(8):
                zbuf[r, pl.ds(j * 16, 16)] = jnp.zeros((16,), jnp.float32)
            return c

        lax.fori_loop(0, ZR, zrow, 0)

        def zero_acc():
            nq = jnp.where(is_last, (STRIPE + 16) // ZR, STRIPE // ZR)

            def zq(q, c):
                pltpu.sync_copy(zbuf, acc.at[pl.ds(t * STRIPE + q * ZR, ZR)])
                return c

            lax.fori_loop(0, nq, zq, 0)

        def run_pass(h, fwd, tab_ref, out_ref):
            nblk = jnp.where(is_last, 200, 320)
            tbase = t * 20480

            def stage_super(b0):
                pltpu.sync_copy(s_ref.at[pl.ds(tbase + b0 * EB, SB)], ids1)
                pltpu.sync_copy(d_ref.at[pl.ds(tbase + b0 * EB, SB)], idd1)

            def transform(b, slot):
                g = b % 8
                gsrc, csrc = (ids1, idd1) if fwd else (idd1, ids1)
                for j in range(EB // 16):
                    jl = pl.ds(j * 16, 16)
                    fl = pl.ds(g * EB + j * 16, 16)
                    idg[slot, 0, jl] = gsrc[fl]
                    idc[slot, 0, jl] = csrc[fl]

            def issue(b, slot):
                pltpu.async_copy(tab_ref.at[idg.at[slot, 0]],
                                 abuf.at[slot], sgs[slot])
                pltpu.async_copy(e_ref.at[pl.ds(h * EE + tbase + b * EB, EB)],
                                 ebuf.at[slot], ses[slot])

            def wait_in(slot):
                pltpu.make_async_copy(tab_ref.at[idg.at[slot, 0]],
                                      abuf.at[slot], sgs[slot]).wait()
                pltpu.make_async_copy(e_ref.at[pl.ds(0, EB)],
                                      ebuf.at[slot], ses[slot]).wait()

            def wait_scatter(slot):
                pltpu.make_async_copy(abuf.at[slot],
                                      acc.at[idc.at[slot, 0]],
                                      sss[slot]).wait()

            stage_super(0)
            transform(0, 0)
            issue(0, 0)

            def body(b, carry):
                for par in range(2):

                    @pl.when((b * 2 + par) < nblk)
                    def _sub():
                        bb = b * 2 + par
                        wait_in(par)

                        def erow(r, c):
                            for j in range(4):
                                jl = pl.ds(j * 16, 16)
                                sg = 1.0 / (1.0 + jnp.exp(-ebuf[par, r, jl]))
                                av = abuf[par, r, pl.ds(h * 64 + j * 16, 16)]
                                abuf[par, r, jl] = sg * av
                                abuf[par, r, pl.ds(64 + j * 16, 16)] = sg
                            return c

                        lax.fori_loop(0, EB, erow, 0)
                        pltpu.async_copy(abuf.at[par],
                                         acc.at[idc.at[par, 0]],
                                         sss[par], add=True)

                        nxt = 1 - par

                        @pl.when(bb + 1 < nblk)
                        def _prefetch():
                            @pl.when(bb + 1 >= 2)
                            def _w1():
                                wait_scatter(nxt)

                            transform(bb + 1, nxt)
                            issue(bb + 1, nxt)

                        @pl.when(jnp.logical_and((bb + 2) % 8 == 0,
                                                 bb + 2 < nblk))
                        def _stage():
                            stage_super(bb + 2)

                return carry

            lax.fori_loop(0, 160, body, 0)
            wait_scatter(0)
            wait_scatter(1)
            plsc.subcore_barrier()
            pltpu.sync_copy(acc.at[pl.ds(t * STRIPE, STRIPE)],
                            out_ref.at[pl.ds(h * NN + t * STRIPE, STRIPE)])

            @pl.when(is_last)
            def _tail():
                pltpu.sync_copy(
                    acc.at[pl.ds(SC_TILES * STRIPE, NN - SC_TILES * STRIPE)],
                    out_ref.at[pl.ds(h * NN + SC_TILES * STRIPE,
                                     NN - SC_TILES * STRIPE)])

        zero_acc()
        plsc.subcore_barrier()
        for h in range(2):
            @pl.when(core == 0)
            def _fwd():
                run_pass(h, True, a2_ref, f_o)

            @pl.when(core == 1)
            def _bwd():
                run_pass(h, False, a3_ref, b_o)

            if h == 0:
                plsc.subcore_barrier()
                zero_acc()
                plsc.subcore_barrier()

    return k(e_fl, a2, a3, src1d, dst1d)


def _tc_mlp(x, w1, b1, w2, b2, block_rows):
    rows, din = x.shape
    dout = w2.shape[1]

    def body(x_ref, w1_ref, b1_ref, w2_ref, b2_ref, o_ref):
        hval = jnp.maximum(x_ref[...] @ w1_ref[...] + b1_ref[...], 0.0)
        o_ref[...] = hval @ w2_ref[...] + b2_ref[...]

    return pl.pallas_call(
        body,
        grid=(rows // block_rows,),
        in_specs=[
            pl.BlockSpec((block_rows, din), lambda i: (i, 0)),
            pl.BlockSpec(w1.shape, lambda i: (0, 0)),
            pl.BlockSpec((1, b1.shape[-1]), lambda i: (0, 0)),
            pl.BlockSpec(w2.shape, lambda i: (0, 0)),
            pl.BlockSpec((1, dout), lambda i: (0, 0)),
        ],
        out_specs=pl.BlockSpec((block_rows, dout), lambda i: (i, 0)),
        out_shape=jax.ShapeDtypeStruct((rows, dout), jnp.float32),
    )(x, w1, b1[None, :], w2, b2[None, :])


def _tc_mlp_chunked_out(x, w1, b1, w2, b2, block_rows):
    rows, din = x.shape

    def body(x_ref, w1_ref, b1_ref, w2_ref, b2_ref, o_ref):
        hval = jnp.maximum(x_ref[...] @ w1_ref[...] + b1_ref[...], 0.0)
        y = hval @ w2_ref[...] + b2_ref[...]
        for c in range(NCH):
            o_ref[c] = y[:, c * CH:(c + 1) * CH]

    return pl.pallas_call(
        body,
        grid=(rows // block_rows,),
        in_specs=[
            pl.BlockSpec((block_rows, din), lambda i: (i, 0)),
            pl.BlockSpec(w1.shape, lambda i: (0, 0)),
            pl.BlockSpec((1, b1.shape[-1]), lambda i: (0, 0)),
            pl.BlockSpec(w2.shape, lambda i: (0, 0)),
            pl.BlockSpec((1, HID), lambda i: (0, 0)),
        ],
        out_specs=pl.BlockSpec((NCH, block_rows, CH), lambda i: (0, i, 0)),
        out_shape=jax.ShapeDtypeStruct((NCH, rows, CH), jnp.float32),
    )(x, w1, b1[None, :], w2, b2[None, :])


def _tc_node_mm(h, wcat, bcat):
    block = 2000

    def body(h_ref, w_ref, b_ref, a1_ref, a2_ref, a3_ref, b1_ref, b2_ref):
        hw = h_ref[...] @ w_ref[...] + b_ref[...]
        a1_ref[...] = hw[:, 0:128]
        a2_ref[...] = hw[:, 128:256]
        a3_ref[...] = hw[:, 256:384]
        b1_ref[...] = hw[:, 384:512]
        b2_ref[...] = hw[:, 512:640]

    ospec = pl.BlockSpec((block, HID), lambda i: (i, 0))
    oshape = jax.ShapeDtypeStruct((NN, HID), jnp.float32)
    return pl.pallas_call(
        body,
        grid=(NN // block,),
        in_specs=[
            pl.BlockSpec((block, HID), lambda i: (i, 0)),
            pl.BlockSpec((HID, 5 * HID), lambda i: (0, 0)),
            pl.BlockSpec((1, 5 * HID), lambda i: (0, 0)),
        ],
        out_specs=[ospec] * 5,
        out_shape=[oshape] * 5,
    )(h, wcat, bcat[None, :])


def _tc_chunked_matmul(e_st, w, b):
    dout = w.shape[1]

    def body(e_ref, w_ref, b_ref, o_ref):
        x = jnp.concatenate([e_ref[c] for c in range(NCH)], axis=-1)
        o_ref[...] = x @ w_ref[...] + b_ref[...]

    return pl.pallas_call(
        body,
        grid=(EE // EBT,),
        in_specs=[
            pl.BlockSpec((NCH, EBT, CH), lambda i: (0, i, 0)),
            pl.BlockSpec((HID, dout), lambda i: (0, 0)),
            pl.BlockSpec((1, dout), lambda i: (0, 0)),
        ],
        out_specs=pl.BlockSpec((EBT, dout), lambda i: (i, 0)),
        out_shape=jax.ShapeDtypeStruct((EE, dout), jnp.float32),
    )(e_st, w, b[None, :])


def _layer_norm_rows(v, g, b):
    mu = jnp.mean(v, axis=-1, keepdims=True)
    var = jnp.mean((v - mu) ** 2, axis=-1, keepdims=True)
    return (v - mu) * jax.lax.rsqrt(var + 1e-5) * g + b


def _tc_edge_update(gsum, b3e, e_st, ln_g, ln_b):
    def body(g_ref, b3_ref, e_ref, lng_ref, lnb_ref, o_ref):
        e_in = jnp.concatenate([e_ref[c] for c in range(NCH)], axis=-1)
        e_hat = _layer_norm_rows(g_ref[...] + b3_ref[...],
                                 lng_ref[...], lnb_ref[...])
        e_new = e_in + jnp.maximum(e_hat, 0.0)
        for c in range(NCH):
            o_ref[c] = e_new[:, c * CH:(c + 1) * CH]

    return pl.pallas_call(
        body,
        grid=(EE // EBT,),
        in_specs=[
            pl.BlockSpec((EBT, HID), lambda i: (i, 0)),
            pl.BlockSpec((EBT, HID), lambda i: (i, 0)),
            pl.BlockSpec((NCH, EBT, CH), lambda i: (0, i, 0)),
            pl.BlockSpec((1, HID), lambda i: (0, 0)),
            pl.BlockSpec((1, HID), lambda i: (0, 0)),
        ],
        out_specs=pl.BlockSpec((NCH, EBT, CH), lambda i: (0, i, 0)),
        out_shape=jax.ShapeDtypeStruct((NCH, EE, CH), jnp.float32),
    )(gsum, b3e, e_st, ln_g[None, :], ln_b[None, :])


def _tc_node_update(h_in, a1h, f_st, b_st, ln_g, ln_b):
    block = 2000

    def body(h_ref, a1_ref, f_ref, b_ref, lng_ref, lnb_ref, o_ref):
        nf = jnp.concatenate([f_ref[0][:, 0:64], f_ref[1][:, 0:64]], axis=-1)
        df = jnp.concatenate([f_ref[0][:, 64:128], f_ref[1][:, 64:128]], axis=-1)
        nb = jnp.concatenate([b_ref[0][:, 0:64], b_ref[1][:, 0:64]], axis=-1)
        db = jnp.concatenate([b_ref[0][:, 64:128], b_ref[1][:, 64:128]], axis=-1)
        h_hat = a1_ref[...] + nf / (df + 1e-6) + nb / (db + 1e-6)
        h_hat = jnp.maximum(
            _layer_norm_rows(h_hat, lng_ref[...], lnb_ref[...]), 0.0)
        o_ref[...] = h_ref[...] + h_hat

    st = pl.BlockSpec((2, block, HID), lambda i: (0, i, 0))
    return pl.pallas_call(
        body,
        grid=(NN // block,),
        in_specs=[
            pl.BlockSpec((block, HID), lambda i: (i, 0)),
            pl.BlockSpec((block, HID), lambda i: (i, 0)),
            st, st,
            pl.BlockSpec((1, HID), lambda i: (0, 0)),
            pl.BlockSpec((1, HID), lambda i: (0, 0)),
        ],
        out_specs=pl.BlockSpec((block, HID), lambda i: (i, 0)),
        out_shape=jax.ShapeDtypeStruct((NN, HID), jnp.float32),
    )(h_in, a1h, f_st, b_st, ln_g[None, :], ln_b[None, :])


def _tc_pred_node(h, w1ab):
    block = 2000

    def body(h_ref, w_ref, o_ref):
        o_ref[...] = h_ref[...] @ w_ref[...]

    return pl.pallas_call(
        body,
        grid=(NN // block,),
        in_specs=[
            pl.BlockSpec((block, HID), lambda i: (i, 0)),
            pl.BlockSpec((HID, HID), lambda i: (0, 0)),
        ],
        out_specs=pl.BlockSpec((block, HID), lambda i: (i, 0)),
        out_shape=jax.ShapeDtypeStruct((NN, HID), jnp.float32),
    )(h, w1ab)


def _tc_score_fin(gz, ec, w2, b2):
    def body(gz_ref, ec_ref, w2_ref, b2_ref, o_ref):
        z = jnp.maximum(gz_ref[...] + ec_ref[...], 0.0)
        o_ref[...] = jnp.sum(z * w2_ref[...], axis=-1, keepdims=True) + b2_ref[...]

    return pl.pallas_call(
        body,
        grid=(EE // EBT,),
        in_specs=[
            pl.BlockSpec((EBT, 64), lambda i: (i, 0)),
            pl.BlockSpec((EBT, 64), lambda i: (i, 0)),
            pl.BlockSpec((1, 64), lambda i: (0, 0)),
            pl.BlockSpec((1, 1), lambda i: (0, 0)),
        ],
        out_specs=pl.BlockSpec((EBT, 1), lambda i: (i, 0)),
        out_shape=jax.ShapeDtypeStruct((EE, 1), jnp.float32),
    )(gz, ec, w2[None, :], b2[None, None, 0])


def kernel(x, e, edge_index, params):
    p = params
    src = edge_index[0]
    dst = edge_index[1]

    h = _tc_mlp(x, p['ne_W1'], p['ne_b1'], p['ne_W2'], p['ne_b2'], 2000)
    e_st = _tc_mlp_chunked_out(e, p['ee_W1'], p['ee_b1'], p['ee_W2'],
                               p['ee_b2'], EBT)

    b3e_next = _tc_chunked_matmul(e_st, p['B3_W'][0], p['B3_b'][0])
    ec = None
    w1 = p['pred_W1']
    for i in range(LAYERS):
        wcat = jnp.concatenate(
            [p['A1_W'][i], p['A2_W'][i], p['A3_W'][i],
             p['B1_W'][i], p['B2_W'][i]], axis=1)
        bcat = jnp.concatenate(
            [p['A1_b'][i], p['A2_b'][i], p['A3_b'][i],
             p['B1_b'][i], p['B2_b'][i]], axis=0)
        a1h, a2h, a3h, b1h, b2h = _tc_node_mm(h, wcat, bcat)
        gsum = _sc_gather2add(b1h, b2h, src, dst, HID)
        e_st = _tc_edge_update(gsum, b3e_next, e_st,
                               p['ln_e_g'][i], p['ln_e_b'][i])
        if i + 1 < LAYERS:
            b3e_next = _tc_chunked_matmul(e_st, p['B3_W'][i + 1],
                                          p['B3_b'][i + 1])
        else:
            ec = _tc_chunked_matmul(e_st, w1[2 * HID:3 * HID], p['pred_b1'])
        e_fl = e_st.reshape(NCH * EE, CH)
        f_fl, b_fl = _sc_agg(e_fl, a2h, a3h, src, dst)
        h = _tc_node_update(h, a1h, f_fl.reshape(2, NN, HID),
                            b_fl.reshape(2, NN, HID),
                            p['ln_h_g'][i], p['ln_h_b'][i])

    hab = _tc_pred_node(h, w1[0:2 * HID].reshape(2, HID, 64)
                        .transpose(1, 0, 2).reshape(HID, HID))
    gz = _sc_gather_head(hab, src, dst)
    return _tc_score_fin(gz, ec, p['pred_W2'][:, 0], p['pred_b2'])


# speedup vs baseline: 3.5713x; 1.0711x over previous
import functools

import jax
import jax.numpy as jnp
from jax import lax
from jax.experimental import pallas as pl
from jax.experimental.pallas import tpu as pltpu
from jax.experimental.pallas import tpu_sc as plsc

NN = 10000
EE = 320000
HID = 128
NCH = 2
CH = 64
LAYERS = 4
SC_CORES = 2
SC_TILES = 16

EBT = 2000


def _mesh():
    return plsc.VectorSubcoreMesh(
        core_axis_name="c", subcore_axis_name="s",
        num_cores=SC_CORES, num_subcores=SC_TILES)


def _sc_gather_kernel(ta, tb, src1d, dst1d, head):
    EB = 128
    SB = 8 * EB
    DO = 64 if head else HID

    scratch = [
        pltpu.VMEM((SB,), jnp.int32),
        pltpu.VMEM((SB,), jnp.int32),
        pltpu.VMEM((2, 1, EB), jnp.int32),
        pltpu.VMEM((2, 1, EB), jnp.int32),
        pltpu.VMEM((2, EB, HID), jnp.float32),
        pltpu.VMEM((2, EB, HID), jnp.float32),
        pltpu.SemaphoreType.DMA,
        pltpu.SemaphoreType.DMA,
        pltpu.SemaphoreType.DMA,
        pltpu.SemaphoreType.DMA,
        pltpu.SemaphoreType.DMA,
        pltpu.SemaphoreType.DMA,
    ]
    if head:
        scratch.insert(6, pltpu.VMEM((2, EB, 64), jnp.float32))

    @functools.partial(
        pl.kernel,
        out_type=jax.ShapeDtypeStruct((EE, DO), jnp.float32),
        mesh=_mesh(),
        scratch_types=scratch,
    )
    def k(ta_ref, tb_ref, s_ref, d_ref, out_ref, *scr):
        if head:
            ids1, idd1, iga, igb, abuf, bbuf, obuf = scr[:7]
            sems = scr[7:]
        else:
            ids1, idd1, iga, igb, abuf, bbuf = scr[:6]
            obuf = abuf
            sems = scr[6:]
        ga = sems[0:2]
        gb = sems[2:4]
        wr = sems[4:6]
        w = lax.axis_index("s") * SC_CORES + lax.axis_index("c")
        nblk = jnp.where(w == 31, 20, 80)
        wbase = w * 10240

        def stage_super(b0):
            pltpu.sync_copy(s_ref.at[pl.ds(wbase + b0 * EB, SB)], ids1)
            pltpu.sync_copy(d_ref.at[pl.ds(wbase + b0 * EB, SB)], idd1)

        def transform(b, slot):
            g = b % 8
            for j in range(EB // 16):
                jl = pl.ds(j * 16, 16)
                fl = pl.ds(g * EB + j * 16, 16)
                iga[slot, 0, jl] = ids1[fl]
                igb[slot, 0, jl] = idd1[fl]

        def issue(b, slot):
            pltpu.async_copy(ta_ref.at[iga.at[slot, 0]], abuf.at[slot], ga[slot])
            pltpu.async_copy(tb_ref.at[igb.at[slot, 0]], bbuf.at[slot], gb[slot])

        def wait_in(slot):
            pltpu.make_async_copy(ta_ref.at[iga.at[slot, 0]],
                                  abuf.at[slot], ga[slot]).wait()
            pltpu.make_async_copy(tb_ref.at[igb.at[slot, 0]],
                                  bbuf.at[slot], gb[slot]).wait()

        def wait_write(slot):
            pltpu.make_async_copy(obuf.at[slot],
                                  out_ref.at[pl.ds(0, EB)], wr[slot]).wait()

        stage_super(0)
        transform(0, 0)
        issue(0, 0)

        def body(b, carry):
            for par in range(2):

                @pl.when((b * 2 + par) < nblk)
                def _sub():
                    bb = b * 2 + par
                    wait_in(par)

                    def add_row(r, c):
                        if head:
                            for j in range(4):
                                obuf[par, r, pl.ds(j * 16, 16)] = (
                                    abuf[par, r, pl.ds(j * 16, 16)]
                                    + bbuf[par, r, pl.ds(64 + j * 16, 16)])
                        else:
                            for j in range(8):
                                jl = pl.ds(j * 16, 16)
                                abuf[par, r, jl] = (abuf[par, r, jl]
                                                    + bbuf[par, r, jl])
                        return c

                    lax.fori_loop(0, EB, add_row, 0)
                    pltpu.async_copy(
                        obuf.at[par],
                        out_ref.at[pl.ds(wbase + bb * EB, EB)], wr[par])
                    nxt = 1 - par

                    @pl.when(bb + 1 < nblk)
                    def _prefetch():
                        @pl.when(bb + 1 >= 2)
                        def _w1():
                            wait_write(nxt)

                        transform(bb + 1, nxt)
                        issue(bb + 1, nxt)

                    @pl.when(jnp.logical_and((bb + 2) % 8 == 0,
                                             bb + 2 < nblk))
                    def _stage():
                        stage_super(bb + 2)

            return carry

        lax.fori_loop(0, 40, body, 0)
        wait_write(0)
        wait_write(1)

    return k(ta, tb, src1d, dst1d)


def _sc_gather2add(ta, tb, src1d, dst1d, D):
    return _sc_gather_kernel(ta, tb, src1d, dst1d, head=False)


def _sc_gather_head(hab, src1d, dst1d):
    return _sc_gather_kernel(hab, hab, src1d, dst1d, head=True)


def _sc_agg(e_fl, a2, a3, src1d, dst1d):
    EB = 64
    SB = 8 * EB
    STRIPE = 624
    ZR = 8

    out2 = [jax.ShapeDtypeStruct((2 * NN, HID), jnp.float32)] * 2

    @functools.partial(
        pl.kernel,
        out_type=out2,
        mesh=_mesh(),
        scratch_types=[
            pltpu.VMEM((SB,), jnp.int32),
            pltpu.VMEM((SB,), jnp.int32),
            pltpu.VMEM((2, 1, EB), jnp.int32),
            pltpu.VMEM((2, 1, EB), jnp.int32),
            pltpu.VMEM((2, EB, 64), jnp.float32),
            pltpu.VMEM((2, EB, HID), jnp.float32),
            pltpu.VMEM((ZR, HID), jnp.float32),
            pltpu.VMEM_SHARED((NN, HID), jnp.float32),
            pltpu.SemaphoreType.DMA,
            pltpu.SemaphoreType.DMA,
            pltpu.SemaphoreType.DMA,
            pltpu.SemaphoreType.DMA,
            pltpu.SemaphoreType.DMA,
            pltpu.SemaphoreType.DMA,
        ],
    )
    def k(e_ref, a2_ref, a3_ref, s_ref, d_ref, f_o, b_o,
          ids1, idd1, idg, idc, ebuf, abuf, zbuf, acc,
          sg0, sg1, se0, se1, ss0, ss1, *_):
        t = lax.axis_index("s")
        core = lax.axis_index("c")
        is_last = t == SC_TILES - 1
        sgs = (sg0, sg1)
        ses = (se0, se1)
        sss = (ss0, ss1)

        def zrow(r, c):
            for j in range(8):
                zbuf[r, pl.ds(j * 16, 16)] = jnp.zeros((16,), jnp.float32)
            return c

        lax.fori_loop(0, ZR, zrow, 0)

        def zero_acc():
            nq = jnp.where(is_last, (STRIPE + 16) // ZR, STRIPE // ZR)

            def zq(q, c):
                pltpu.sync_copy(zbuf, acc.at[pl.ds(t * STRIPE + q * ZR, ZR)])
                return c

            lax.fori_loop(0, nq, zq, 0)

        def run_pass(h, fwd, tab_ref, out_ref):
            nblk = jnp.where(is_last, 200, 320)
            tbase = t * 20480

            def stage_super(b0):
                pltpu.sync_copy(s_ref.at[pl.ds(tbase + b0 * EB, SB)], ids1)
                pltpu.sync_copy(d_ref.at[pl.ds(tbase + b0 * EB, SB)], idd1)

            def transform(b, slot):
                g = b % 8
                gsrc, csrc = (ids1, idd1) if fwd else (idd1, ids1)
                for j in range(EB // 16):
                    jl = pl.ds(j * 16, 16)
                    fl = pl.ds(g * EB + j * 16, 16)
                    idg[slot, 0, jl] = gsrc[fl]
                    idc[slot, 0, jl] = csrc[fl]

            def issue(b, slot):
                pltpu.async_copy(tab_ref.at[idg.at[slot, 0]],
                                 abuf.at[slot], sgs[slot])
                pltpu.async_copy(e_ref.at[pl.ds(h * EE + tbase + b * EB, EB)],
                                 ebuf.at[slot], ses[slot])

            def wait_in(slot):
                pltpu.make_async_copy(tab_ref.at[idg.at[slot, 0]],
                                      abuf.at[slot], sgs[slot]).wait()
                pltpu.make_async_copy(e_ref.at[pl.ds(0, EB)],
                                      ebuf.at[slot], ses[slot]).wait()

            def wait_scatter(slot):
                pltpu.make_async_copy(abuf.at[slot],
                                      acc.at[idc.at[slot, 0]],
                                      sss[slot]).wait()

            stage_super(0)
            transform(0, 0)
            issue(0, 0)

            def body(b, carry):
                for par in range(2):

                    @pl.when((b * 2 + par) < nblk)
                    def _sub():
                        bb = b * 2 + par
                        wait_in(par)

                        def erow(r, c):
                            for j in range(4):
                                jl = pl.ds(j * 16, 16)
                                sg = 1.0 / (1.0 + jnp.exp(-ebuf[par, r, jl]))
                                av = abuf[par, r, pl.ds(h * 64 + j * 16, 16)]
                                abuf[par, r, jl] = sg * av
                                abuf[par, r, pl.ds(64 + j * 16, 16)] = sg
                            return c

                        lax.fori_loop(0, EB, erow, 0)
                        pltpu.async_copy(abuf.at[par],
                                         acc.at[idc.at[par, 0]],
                                         sss[par], add=True)

                        nxt = 1 - par

                        @pl.when(bb + 1 < nblk)
                        def _prefetch():
                            @pl.when(bb + 1 >= 2)
                            def _w1():
                                wait_scatter(nxt)

                            transform(bb + 1, nxt)
                            issue(bb + 1, nxt)

                        @pl.when(jnp.logical_and((bb + 2) % 8 == 0,
                                                 bb + 2 < nblk))
                        def _stage():
                            stage_super(bb + 2)

                return carry

            lax.fori_loop(0, 160, body, 0)
            wait_scatter(0)
            wait_scatter(1)
            plsc.subcore_barrier()
            pltpu.sync_copy(acc.at[pl.ds(t * STRIPE, STRIPE)],
                            out_ref.at[pl.ds(h * NN + t * STRIPE, STRIPE)])

            @pl.when(is_last)
            def _tail():
                pltpu.sync_copy(
                    acc.at[pl.ds(SC_TILES * STRIPE, NN - SC_TILES * STRIPE)],
                    out_ref.at[pl.ds(h * NN + SC_TILES * STRIPE,
                                     NN - SC_TILES * STRIPE)])

        zero_acc()
        plsc.subcore_barrier()
        for h in range(2):
            @pl.when(core == 0)
            def _fwd():
                run_pass(h, True, a2_ref, f_o)

            @pl.when(core == 1)
            def _bwd():
                run_pass(h, False, a3_ref, b_o)

            if h == 0:
                plsc.subcore_barrier()
                zero_acc()
                plsc.subcore_barrier()

    return k(e_fl, a2, a3, src1d, dst1d)


def _tc_mlp(x, w1, b1, w2, b2, block_rows):
    rows, din = x.shape
    dout = w2.shape[1]

    def body(x_ref, w1_ref, b1_ref, w2_ref, b2_ref, o_ref):
        hval = jnp.maximum(x_ref[...] @ w1_ref[...] + b1_ref[...], 0.0)
        o_ref[...] = hval @ w2_ref[...] + b2_ref[...]

    return pl.pallas_call(
        body,
        grid=(rows // block_rows,),
        in_specs=[
            pl.BlockSpec((block_rows, din), lambda i: (i, 0)),
            pl.BlockSpec(w1.shape, lambda i: (0, 0)),
            pl.BlockSpec((1, b1.shape[-1]), lambda i: (0, 0)),
            pl.BlockSpec(w2.shape, lambda i: (0, 0)),
            pl.BlockSpec((1, dout), lambda i: (0, 0)),
        ],
        out_specs=pl.BlockSpec((block_rows, dout), lambda i: (i, 0)),
        out_shape=jax.ShapeDtypeStruct((rows, dout), jnp.float32),
    )(x, w1, b1[None, :], w2, b2[None, :])


def _tc_mlp_chunked_out(x, w1, b1, w2, b2, block_rows):
    rows, din = x.shape

    def body(x_ref, w1_ref, b1_ref, w2_ref, b2_ref, o_ref):
        hval = jnp.maximum(x_ref[...] @ w1_ref[...] + b1_ref[...], 0.0)
        y = hval @ w2_ref[...] + b2_ref[...]
        for c in range(NCH):
            o_ref[c] = y[:, c * CH:(c + 1) * CH]

    return pl.pallas_call(
        body,
        grid=(rows // block_rows,),
        in_specs=[
            pl.BlockSpec((block_rows, din), lambda i: (i, 0)),
            pl.BlockSpec(w1.shape, lambda i: (0, 0)),
            pl.BlockSpec((1, b1.shape[-1]), lambda i: (0, 0)),
            pl.BlockSpec(w2.shape, lambda i: (0, 0)),
            pl.BlockSpec((1, HID), lambda i: (0, 0)),
        ],
        out_specs=pl.BlockSpec((NCH, block_rows, CH), lambda i: (0, i, 0)),
        out_shape=jax.ShapeDtypeStruct((NCH, rows, CH), jnp.float32),
    )(x, w1, b1[None, :], w2, b2[None, :])


def _tc_node_mm(h, wcat, bcat):
    block = 2000

    def body(h_ref, w_ref, b_ref, a1_ref, a2_ref, a3_ref, b1_ref, b2_ref):
        hw = h_ref[...] @ w_ref[...] + b_ref[...]
        a1_ref[...] = hw[:, 0:128]
        a2_ref[...] = hw[:, 128:256]
        a3_ref[...] = hw[:, 256:384]
        b1_ref[...] = hw[:, 384:512]
        b2_ref[...] = hw[:, 512:640]

    ospec = pl.BlockSpec((block, HID), lambda i: (i, 0))
    oshape = jax.ShapeDtypeStruct((NN, HID), jnp.float32)
    return pl.pallas_call(
        body,
        grid=(NN // block,),
        in_specs=[
            pl.BlockSpec((block, HID), lambda i: (i, 0)),
            pl.BlockSpec((HID, 5 * HID), lambda i: (0, 0)),
            pl.BlockSpec((1, 5 * HID), lambda i: (0, 0)),
        ],
        out_specs=[ospec] * 5,
        out_shape=[oshape] * 5,
    )(h, wcat, bcat[None, :])


def _tc_chunked_matmul(e_st, w, b):
    dout = w.shape[1]

    def body(e_ref, w_ref, b_ref, o_ref):
        x = jnp.concatenate([e_ref[c] for c in range(NCH)], axis=-1)
        o_ref[...] = x @ w_ref[...] + b_ref[...]

    return pl.pallas_call(
        body,
        grid=(EE // EBT,),
        in_specs=[
            pl.BlockSpec((NCH, EBT, CH), lambda i: (0, i, 0)),
            pl.BlockSpec((HID, dout), lambda i: (0, 0)),
            pl.BlockSpec((1, dout), lambda i: (0, 0)),
        ],
        out_specs=pl.BlockSpec((EBT, dout), lambda i: (i, 0)),
        out_shape=jax.ShapeDtypeStruct((EE, dout), jnp.float32),
    )(e_st, w, b[None, :])


def _layer_norm_rows(v, g, b):
    mu = jnp.mean(v, axis=-1, keepdims=True)
    var = jnp.mean((v - mu) ** 2, axis=-1, keepdims=True)
    return (v - mu) * jax.lax.rsqrt(var + 1e-5) * g + b


def _tc_edge_update(gsum, e_st, b3w, b3b, ln_g, ln_b):
    def body(g_ref, e_ref, w_ref, b_ref, lng_ref, lnb_ref, o_ref):
        e_in = jnp.concatenate([e_ref[c] for c in range(NCH)], axis=-1)
        b3e = e_in @ w_ref[...] + b_ref[...]
        e_hat = _layer_norm_rows(g_ref[...] + b3e, lng_ref[...], lnb_ref[...])
        e_new = e_in + jnp.maximum(e_hat, 0.0)
        for c in range(NCH):
            o_ref[c] = e_new[:, c * CH:(c + 1) * CH]

    return pl.pallas_call(
        body,
        grid=(EE // EBT,),
        in_specs=[
            pl.BlockSpec((EBT, HID), lambda i: (i, 0)),
            pl.BlockSpec((NCH, EBT, CH), lambda i: (0, i, 0)),
            pl.BlockSpec((HID, HID), lambda i: (0, 0)),
            pl.BlockSpec((1, HID), lambda i: (0, 0)),
            pl.BlockSpec((1, HID), lambda i: (0, 0)),
            pl.BlockSpec((1, HID), lambda i: (0, 0)),
        ],
        out_specs=pl.BlockSpec((NCH, EBT, CH), lambda i: (0, i, 0)),
        out_shape=jax.ShapeDtypeStruct((NCH, EE, CH), jnp.float32),
    )(gsum, e_st, b3w, b3b[None, :], ln_g[None, :], ln_b[None, :])


def _tc_node_update(h_in, a1h, f_st, b_st, ln_g, ln_b):
    block = 2000

    def body(h_ref, a1_ref, f_ref, b_ref, lng_ref, lnb_ref, o_ref):
        nf = jnp.concatenate([f_ref[0][:, 0:64], f_ref[1][:, 0:64]], axis=-1)
        df = jnp.concatenate([f_ref[0][:, 64:128], f_ref[1][:, 64:128]], axis=-1)
        nb = jnp.concatenate([b_ref[0][:, 0:64], b_ref[1][:, 0:64]], axis=-1)
        db = jnp.concatenate([b_ref[0][:, 64:128], b_ref[1][:, 64:128]], axis=-1)
        h_hat = a1_ref[...] + nf / (df + 1e-6) + nb / (db + 1e-6)
        h_hat = jnp.maximum(
            _layer_norm_rows(h_hat, lng_ref[...], lnb_ref[...]), 0.0)
        o_ref[...] = h_ref[...] + h_hat

    st = pl.BlockSpec((2, block, HID), lambda i: (0, i, 0))
    return pl.pallas_call(
        body,
        grid=(NN // block,),
        in_specs=[
            pl.BlockSpec((block, HID), lambda i: (i, 0)),
            pl.BlockSpec((block, HID), lambda i: (i, 0)),
            st, st,
            pl.BlockSpec((1, HID), lambda i: (0, 0)),
            pl.BlockSpec((1, HID), lambda i: (0, 0)),
        ],
        out_specs=pl.BlockSpec((block, HID), lambda i: (i, 0)),
        out_shape=jax.ShapeDtypeStruct((NN, HID), jnp.float32),
    )(h_in, a1h, f_st, b_st, ln_g[None, :], ln_b[None, :])


def _tc_pred_node(h, w1ab):
    block = 2000

    def body(h_ref, w_ref, o_ref):
        o_ref[...] = h_ref[...] @ w_ref[...]

    return pl.pallas_call(
        body,
        grid=(NN // block,),
        in_specs=[
            pl.BlockSpec((block, HID), lambda i: (i, 0)),
            pl.BlockSpec((HID, HID), lambda i: (0, 0)),
        ],
        out_specs=pl.BlockSpec((block, HID), lambda i: (i, 0)),
        out_shape=jax.ShapeDtypeStruct((NN, HID), jnp.float32),
    )(h, w1ab)


def _tc_score_fin(gz, e_st, w1c, b1, w2, b2):
    def body(gz_ref, e_ref, wc_ref, b1_ref, w2_ref, b2_ref, o_ref):
        e_in = jnp.concatenate([e_ref[c] for c in range(NCH)], axis=-1)
        z = jnp.maximum(gz_ref[...] + e_in @ wc_ref[...] + b1_ref[...], 0.0)
        o_ref[...] = jnp.sum(z * w2_ref[...], axis=-1, keepdims=True) + b2_ref[...]

    return pl.pallas_call(
        body,
        grid=(EE // EBT,),
        in_specs=[
            pl.BlockSpec((EBT, 64), lambda i: (i, 0)),
            pl.BlockSpec((NCH, EBT, CH), lambda i: (0, i, 0)),
            pl.BlockSpec((HID, 64), lambda i: (0, 0)),
            pl.BlockSpec((1, 64), lambda i: (0, 0)),
            pl.BlockSpec((1, 64), lambda i: (0, 0)),
            pl.BlockSpec((1, 1), lambda i: (0, 0)),
        ],
        out_specs=pl.BlockSpec((EBT, 1), lambda i: (i, 0)),
        out_shape=jax.ShapeDtypeStruct((EE, 1), jnp.float32),
    )(gz, e_st, w1c, b1[None, :], w2[None, :], b2[None, None, 0])


def kernel(x, e, edge_index, params):
    p = params
    src = edge_index[0]
    dst = edge_index[1]

    h = _tc_mlp(x, p['ne_W1'], p['ne_b1'], p['ne_W2'], p['ne_b2'], 2000)
    e_st = _tc_mlp_chunked_out(e, p['ee_W1'], p['ee_b1'], p['ee_W2'],
                               p['ee_b2'], EBT)

    w1 = p['pred_W1']
    for i in range(LAYERS):
        wcat = jnp.concatenate(
            [p['A1_W'][i], p['A2_W'][i], p['A3_W'][i],
             p['B1_W'][i], p['B2_W'][i]], axis=1)
        bcat = jnp.concatenate(
            [p['A1_b'][i], p['A2_b'][i], p['A3_b'][i],
             p['B1_b'][i], p['B2_b'][i]], axis=0)
        a1h, a2h, a3h, b1h, b2h = _tc_node_mm(h, wcat, bcat)
        gsum = _sc_gather2add(b1h, b2h, src, dst, HID)
        e_st = _tc_edge_update(gsum, e_st, p['B3_W'][i], p['B3_b'][i],
                               p['ln_e_g'][i], p['ln_e_b'][i])
        e_fl = e_st.reshape(NCH * EE, CH)
        f_fl, b_fl = _sc_agg(e_fl, a2h, a3h, src, dst)
        h = _tc_node_update(h, a1h, f_fl.reshape(2, NN, HID),
                            b_fl.reshape(2, NN, HID),
                            p['ln_h_g'][i], p['ln_h_b'][i])

    hab = _tc_pred_node(h, w1[0:2 * HID].reshape(2, HID, 64)
                        .transpose(1, 0, 2).reshape(HID, HID))
    gz = _sc_gather_head(hab, src, dst)
    return _tc_score_fin(gz, e_st, w1[2 * HID:3 * HID], p['pred_b1'],
                         p['pred_W2'][:, 0], p['pred_b2'])


# speedup vs baseline: 3.6960x; 1.0349x over previous
import functools

import jax
import jax.numpy as jnp
from jax import lax
from jax.experimental import pallas as pl
from jax.experimental.pallas import tpu as pltpu
from jax.experimental.pallas import tpu_sc as plsc

NN = 10000
EE = 320000
HID = 128
NCH = 2
CH = 64
LAYERS = 4
SC_CORES = 2
SC_TILES = 16

EBT = 2000


def _mesh():
    return plsc.VectorSubcoreMesh(
        core_axis_name="c", subcore_axis_name="s",
        num_cores=SC_CORES, num_subcores=SC_TILES)


def _sc_gather_kernel(ta, tb, src1d, dst1d, head):
    EB = 128
    SB = 8 * EB
    DO = 64 if head else HID

    scratch = [
        pltpu.VMEM((SB,), jnp.int32),
        pltpu.VMEM((SB,), jnp.int32),
        pltpu.VMEM((2, 1, EB), jnp.int32),
        pltpu.VMEM((2, 1, EB), jnp.int32),
        pltpu.VMEM((2, EB, HID), jnp.float32),
        pltpu.VMEM((2, EB, HID), jnp.float32),
        pltpu.SemaphoreType.DMA,
        pltpu.SemaphoreType.DMA,
        pltpu.SemaphoreType.DMA,
        pltpu.SemaphoreType.DMA,
        pltpu.SemaphoreType.DMA,
        pltpu.SemaphoreType.DMA,
    ]
    if head:
        scratch.insert(6, pltpu.VMEM((2, EB, 64), jnp.float32))

    @functools.partial(
        pl.kernel,
        out_type=jax.ShapeDtypeStruct((EE, DO), jnp.float32),
        mesh=_mesh(),
        scratch_types=scratch,
    )
    def k(ta_ref, tb_ref, s_ref, d_ref, out_ref, *scr):
        if head:
            ids1, idd1, iga, igb, abuf, bbuf, obuf = scr[:7]
            sems = scr[7:]
        else:
            ids1, idd1, iga, igb, abuf, bbuf = scr[:6]
            obuf = abuf
            sems = scr[6:]
        ga = sems[0:2]
        gb = sems[2:4]
        wr = sems[4:6]
        w = lax.axis_index("s") * SC_CORES + lax.axis_index("c")
        nblk = jnp.where(w == 31, 20, 80)
        wbase = w * 10240

        def stage_super(b0):
            pltpu.sync_copy(s_ref.at[pl.ds(wbase + b0 * EB, SB)], ids1)
            pltpu.sync_copy(d_ref.at[pl.ds(wbase + b0 * EB, SB)], idd1)

        def transform(b, slot):
            g = b % 8
            for j in range(EB // 16):
                jl = pl.ds(j * 16, 16)
                fl = pl.ds(g * EB + j * 16, 16)
                iga[slot, 0, jl] = ids1[fl]
                igb[slot, 0, jl] = idd1[fl]

        def issue(b, slot):
            pltpu.async_copy(ta_ref.at[iga.at[slot, 0]], abuf.at[slot], ga[slot])
            pltpu.async_copy(tb_ref.at[igb.at[slot, 0]], bbuf.at[slot], gb[slot])

        def wait_in(slot):
            pltpu.make_async_copy(ta_ref.at[iga.at[slot, 0]],
                                  abuf.at[slot], ga[slot]).wait()
            pltpu.make_async_copy(tb_ref.at[igb.at[slot, 0]],
                                  bbuf.at[slot], gb[slot]).wait()

        def wait_write(slot):
            pltpu.make_async_copy(obuf.at[slot],
                                  out_ref.at[pl.ds(0, EB)], wr[slot]).wait()

        stage_super(0)
        transform(0, 0)
        issue(0, 0)

        def body(b, carry):
            for par in range(2):

                @pl.when((b * 2 + par) < nblk)
                def _sub():
                    bb = b * 2 + par
                    wait_in(par)

                    def add_row(r, c):
                        if head:
                            for j in range(4):
                                obuf[par, r, pl.ds(j * 16, 16)] = (
                                    abuf[par, r, pl.ds(j * 16, 16)]
                                    + bbuf[par, r, pl.ds(64 + j * 16, 16)])
                        else:
                            for j in range(8):
                                jl = pl.ds(j * 16, 16)
                                abuf[par, r, jl] = (abuf[par, r, jl]
                                                    + bbuf[par, r, jl])
                        return c

                    lax.fori_loop(0, EB, add_row, 0)
                    pltpu.async_copy(
                        obuf.at[par],
                        out_ref.at[pl.ds(wbase + bb * EB, EB)], wr[par])
                    nxt = 1 - par

                    @pl.when(bb + 1 < nblk)
                    def _prefetch():
                        @pl.when(bb + 1 >= 2)
                        def _w1():
                            wait_write(nxt)

                        transform(bb + 1, nxt)
                        issue(bb + 1, nxt)

                    @pl.when(jnp.logical_and((bb + 2) % 8 == 0,
                                             bb + 2 < nblk))
                    def _stage():
                        stage_super(bb + 2)

            return carry

        lax.fori_loop(0, 40, body, 0)
        wait_write(0)
        wait_write(1)

    return k(ta, tb, src1d, dst1d)


def _sc_gather2add(ta, tb, src1d, dst1d, D):
    return _sc_gather_kernel(ta, tb, src1d, dst1d, head=False)


def _sc_gather_head(hab, src1d, dst1d):
    return _sc_gather_kernel(hab, hab, src1d, dst1d, head=True)


def _sc_agg(e_fl, a2, a3, src1d, dst1d):
    EB = 80
    SB = 8 * EB
    STRIPE = 624
    ZR = 8

    out2 = [jax.ShapeDtypeStruct((2 * NN, HID), jnp.float32)] * 2

    @functools.partial(
        pl.kernel,
        out_type=out2,
        mesh=_mesh(),
        scratch_types=[
            pltpu.VMEM((SB,), jnp.int32),
            pltpu.VMEM((SB,), jnp.int32),
            pltpu.VMEM((2, 1, EB), jnp.int32),
            pltpu.VMEM((2, 1, EB), jnp.int32),
            pltpu.VMEM((2, EB, 64), jnp.float32),
            pltpu.VMEM((2, EB, HID), jnp.float32),
            pltpu.VMEM((ZR, HID), jnp.float32),
            pltpu.VMEM_SHARED((NN, HID), jnp.float32),
            pltpu.SemaphoreType.DMA,
            pltpu.SemaphoreType.DMA,
            pltpu.SemaphoreType.DMA,
            pltpu.SemaphoreType.DMA,
            pltpu.SemaphoreType.DMA,
            pltpu.SemaphoreType.DMA,
        ],
    )
    def k(e_ref, a2_ref, a3_ref, s_ref, d_ref, f_o, b_o,
          ids1, idd1, idg, idc, ebuf, abuf, zbuf, acc,
          sg0, sg1, se0, se1, ss0, ss1, *_):
        t = lax.axis_index("s")
        core = lax.axis_index("c")
        is_last = t == SC_TILES - 1
        sgs = (sg0, sg1)
        ses = (se0, se1)
        sss = (ss0, ss1)

        def zrow(r, c):
            for j in range(8):
                zbuf[r, pl.ds(j * 16, 16)] = jnp.zeros((16,), jnp.float32)
            return c

        lax.fori_loop(0, ZR, zrow, 0)

        def zero_acc():
            nq = jnp.where(is_last, (STRIPE + 16) // ZR, STRIPE // ZR)

            def zq(q, c):
                pltpu.sync_copy(zbuf, acc.at[pl.ds(t * STRIPE + q * ZR, ZR)])
                return c

            lax.fori_loop(0, nq, zq, 0)

        def run_pass(h, fwd, tab_ref, out_ref):
            nblk = jnp.where(is_last, 160, 256)
            tbase = t * 20480

            def stage_super(b0):
                pltpu.sync_copy(s_ref.at[pl.ds(tbase + b0 * EB, SB)], ids1)
                pltpu.sync_copy(d_ref.at[pl.ds(tbase + b0 * EB, SB)], idd1)

            def transform(b, slot):
                g = b % 8
                gsrc, csrc = (ids1, idd1) if fwd else (idd1, ids1)
                for j in range(EB // 16):
                    jl = pl.ds(j * 16, 16)
                    fl = pl.ds(g * EB + j * 16, 16)
                    idg[slot, 0, jl] = gsrc[fl]
                    idc[slot, 0, jl] = csrc[fl]

            def issue(b, slot):
                pltpu.async_copy(tab_ref.at[idg.at[slot, 0]],
                                 abuf.at[slot], sgs[slot])
                pltpu.async_copy(e_ref.at[pl.ds(h * EE + tbase + b * EB, EB)],
                                 ebuf.at[slot], ses[slot])

            def wait_in(slot):
                pltpu.make_async_copy(tab_ref.at[idg.at[slot, 0]],
                                      abuf.at[slot], sgs[slot]).wait()
                pltpu.make_async_copy(e_ref.at[pl.ds(0, EB)],
                                      ebuf.at[slot], ses[slot]).wait()

            def wait_scatter(slot):
                pltpu.make_async_copy(abuf.at[slot],
                                      acc.at[idc.at[slot, 0]],
                                      sss[slot]).wait()

            stage_super(0)
            transform(0, 0)
            issue(0, 0)

            def body(b, carry):
                for par in range(2):

                    @pl.when((b * 2 + par) < nblk)
                    def _sub():
                        bb = b * 2 + par
                        wait_in(par)

                        def erow(r, c):
                            for j in range(4):
                                jl = pl.ds(j * 16, 16)
                                sg = 1.0 / (1.0 + jnp.exp(-ebuf[par, r, jl]))
                                av = abuf[par, r, pl.ds(h * 64 + j * 16, 16)]
                                abuf[par, r, jl] = sg * av
                                abuf[par, r, pl.ds(64 + j * 16, 16)] = sg
                            return c

                        lax.fori_loop(0, EB, erow, 0)
                        pltpu.async_copy(abuf.at[par],
                                         acc.at[idc.at[par, 0]],
                                         sss[par], add=True)

                        nxt = 1 - par

                        @pl.when(bb + 1 < nblk)
                        def _prefetch():
                            @pl.when(bb + 1 >= 2)
                            def _w1():
                                wait_scatter(nxt)

                            transform(bb + 1, nxt)
                            issue(bb + 1, nxt)

                        @pl.when(jnp.logical_and((bb + 2) % 8 == 0,
                                                 bb + 2 < nblk))
                        def _stage():
                            stage_super(bb + 2)

                return carry

            lax.fori_loop(0, 128, body, 0)
            wait_scatter(0)
            wait_scatter(1)
            plsc.subcore_barrier()
            pltpu.sync_copy(acc.at[pl.ds(t * STRIPE, STRIPE)],
                            out_ref.at[pl.ds(h * NN + t * STRIPE, STRIPE)])

            @pl.when(is_last)
            def _tail():
                pltpu.sync_copy(
                    acc.at[pl.ds(SC_TILES * STRIPE, NN - SC_TILES * STRIPE)],
                    out_ref.at[pl.ds(h * NN + SC_TILES * STRIPE,
                                     NN - SC_TILES * STRIPE)])

        zero_acc()
        plsc.subcore_barrier()
        for h in range(2):
            @pl.when(core == 0)
            def _fwd():
                run_pass(h, True, a2_ref, f_o)

            @pl.when(core == 1)
            def _bwd():
                run_pass(h, False, a3_ref, b_o)

            if h == 0:
                plsc.subcore_barrier()
                zero_acc()
                plsc.subcore_barrier()

    return k(e_fl, a2, a3, src1d, dst1d)


def _tc_mlp(x, w1, b1, w2, b2, block_rows):
    rows, din = x.shape
    dout = w2.shape[1]

    def body(x_ref, w1_ref, b1_ref, w2_ref, b2_ref, o_ref):
        hval = jnp.maximum(x_ref[...] @ w1_ref[...] + b1_ref[...], 0.0)
        o_ref[...] = hval @ w2_ref[...] + b2_ref[...]

    return pl.pallas_call(
        body,
        grid=(rows // block_rows,),
        in_specs=[
            pl.BlockSpec((block_rows, din), lambda i: (i, 0)),
            pl.BlockSpec(w1.shape, lambda i: (0, 0)),
            pl.BlockSpec((1, b1.shape[-1]), lambda i: (0, 0)),
            pl.BlockSpec(w2.shape, lambda i: (0, 0)),
            pl.BlockSpec((1, dout), lambda i: (0, 0)),
        ],
        out_specs=pl.BlockSpec((block_rows, dout), lambda i: (i, 0)),
        out_shape=jax.ShapeDtypeStruct((rows, dout), jnp.float32),
    )(x, w1, b1[None, :], w2, b2[None, :])


def _tc_mlp_chunked_out(x, w1, b1, w2, b2, block_rows):
    rows, din = x.shape

    def body(x_ref, w1_ref, b1_ref, w2_ref, b2_ref, o_ref):
        hval = jnp.maximum(x_ref[...] @ w1_ref[...] + b1_ref[...], 0.0)
        y = hval @ w2_ref[...] + b2_ref[...]
        for c in range(NCH):
            o_ref[c] = y[:, c * CH:(c + 1) * CH]

    return pl.pallas_call(
        body,
        grid=(rows // block_rows,),
        in_specs=[
            pl.BlockSpec((block_rows, din), lambda i: (i, 0)),
            pl.BlockSpec(w1.shape, lambda i: (0, 0)),
            pl.BlockSpec((1, b1.shape[-1]), lambda i: (0, 0)),
            pl.BlockSpec(w2.shape, lambda i: (0, 0)),
            pl.BlockSpec((1, HID), lambda i: (0, 0)),
        ],
        out_specs=pl.BlockSpec((NCH, block_rows, CH), lambda i: (0, i, 0)),
        out_shape=jax.ShapeDtypeStruct((NCH, rows, CH), jnp.float32),
    )(x, w1, b1[None, :], w2, b2[None, :])


def _tc_node_mm(h, wcat, bcat):
    block = 2000

    def body(h_ref, w_ref, b_ref, a1_ref, a2_ref, a3_ref, b1_ref, b2_ref):
        hw = h_ref[...] @ w_ref[...] + b_ref[...]
        a1_ref[...] = hw[:, 0:128]
        a2_ref[...] = hw[:, 128:256]
        a3_ref[...] = hw[:, 256:384]
        b1_ref[...] = hw[:, 384:512]
        b2_ref[...] = hw[:, 512:640]

    ospec = pl.BlockSpec((block, HID), lambda i: (i, 0))
    oshape = jax.ShapeDtypeStruct((NN, HID), jnp.float32)
    return pl.pallas_call(
        body,
        grid=(NN // block,),
        in_specs=[
            pl.BlockSpec((block, HID), lambda i: (i, 0)),
            pl.BlockSpec((HID, 5 * HID), lambda i: (0, 0)),
            pl.BlockSpec((1, 5 * HID), lambda i: (0, 0)),
        ],
        out_specs=[ospec] * 5,
        out_shape=[oshape] * 5,
    )(h, wcat, bcat[None, :])


def _tc_chunked_matmul(e_st, w, b):
    dout = w.shape[1]

    def body(e_ref, w_ref, b_ref, o_ref):
        x = jnp.concatenate([e_ref[c] for c in range(NCH)], axis=-1)
        o_ref[...] = x @ w_ref[...] + b_ref[...]

    return pl.pallas_call(
        body,
        grid=(EE // EBT,),
        in_specs=[
            pl.BlockSpec((NCH, EBT, CH), lambda i: (0, i, 0)),
            pl.BlockSpec((HID, dout), lambda i: (0, 0)),
            pl.BlockSpec((1, dout), lambda i: (0, 0)),
        ],
        out_specs=pl.BlockSpec((EBT, dout), lambda i: (i, 0)),
        out_shape=jax.ShapeDtypeStruct((EE, dout), jnp.float32),
    )(e_st, w, b[None, :])


def _layer_norm_rows(v, g, b):
    mu = jnp.mean(v, axis=-1, keepdims=True)
    var = jnp.mean((v - mu) ** 2, axis=-1, keepdims=True)
    return (v - mu) * jax.lax.rsqrt(var + 1e-5) * g + b


def _tc_edge_update(gsum, e_st, b3w, b3b, ln_g, ln_b):
    def body(g_ref, e_ref, w_ref, b_ref, lng_ref, lnb_ref, o_ref):
        e_in = jnp.concatenate([e_ref[c] for c in range(NCH)], axis=-1)
        b3e = e_in @ w_ref[...] + b_ref[...]
        e_hat = _layer_norm_rows(g_ref[...] + b3e, lng_ref[...], lnb_ref[...])
        e_new = e_in + jnp.maximum(e_hat, 0.0)
        for c in range(NCH):
            o_ref[c] = e_new[:, c * CH:(c + 1) * CH]

    return pl.pallas_call(
        body,
        grid=(EE // EBT,),
        in_specs=[
            pl.BlockSpec((EBT, HID), lambda i: (i, 0)),
            pl.BlockSpec((NCH, EBT, CH), lambda i: (0, i, 0)),
            pl.BlockSpec((HID, HID), lambda i: (0, 0)),
            pl.BlockSpec((1, HID), lambda i: (0, 0)),
            pl.BlockSpec((1, HID), lambda i: (0, 0)),
            pl.BlockSpec((1, HID), lambda i: (0, 0)),
        ],
        out_specs=pl.BlockSpec((NCH, EBT, CH), lambda i: (0, i, 0)),
        out_shape=jax.ShapeDtypeStruct((NCH, EE, CH), jnp.float32),
    )(gsum, e_st, b3w, b3b[None, :], ln_g[None, :], ln_b[None, :])


def _tc_node_update(h_in, a1h, f_st, b_st, ln_g, ln_b):
    block = 2000

    def body(h_ref, a1_ref, f_ref, b_ref, lng_ref, lnb_ref, o_ref):
        nf = jnp.concatenate([f_ref[0][:, 0:64], f_ref[1][:, 0:64]], axis=-1)
        df = jnp.concatenate([f_ref[0][:, 64:128], f_ref[1][:, 64:128]], axis=-1)
        nb = jnp.concatenate([b_ref[0][:, 0:64], b_ref[1][:, 0:64]], axis=-1)
        db = jnp.concatenate([b_ref[0][:, 64:128], b_ref[1][:, 64:128]], axis=-1)
        h_hat = a1_ref[...] + nf / (df + 1e-6) + nb / (db + 1e-6)
        h_hat = jnp.maximum(
            _layer_norm_rows(h_hat, lng_ref[...], lnb_ref[...]), 0.0)
        o_ref[...] = h_ref[...] + h_hat

    st = pl.BlockSpec((2, block, HID), lambda i: (0, i, 0))
    return pl.pallas_call(
        body,
        grid=(NN // block,),
        in_specs=[
            pl.BlockSpec((block, HID), lambda i: (i, 0)),
            pl.BlockSpec((block, HID), lambda i: (i, 0)),
            st, st,
            pl.BlockSpec((1, HID), lambda i: (0, 0)),
            pl.BlockSpec((1, HID), lambda i: (0, 0)),
        ],
        out_specs=pl.BlockSpec((block, HID), lambda i: (i, 0)),
        out_shape=jax.ShapeDtypeStruct((NN, HID), jnp.float32),
    )(h_in, a1h, f_st, b_st, ln_g[None, :], ln_b[None, :])


def _tc_pred_node(h, w1ab):
    block = 2000

    def body(h_ref, w_ref, o_ref):
        o_ref[...] = h_ref[...] @ w_ref[...]

    return pl.pallas_call(
        body,
        grid=(NN // block,),
        in_specs=[
            pl.BlockSpec((block, HID), lambda i: (i, 0)),
            pl.BlockSpec((HID, HID), lambda i: (0, 0)),
        ],
        out_specs=pl.BlockSpec((block, HID), lambda i: (i, 0)),
        out_shape=jax.ShapeDtypeStruct((NN, HID), jnp.float32),
    )(h, w1ab)


def _tc_score_fin(gz, e_st, w1c, b1, w2, b2):
    def body(gz_ref, e_ref, wc_ref, b1_ref, w2_ref, b2_ref, o_ref):
        e_in = jnp.concatenate([e_ref[c] for c in range(NCH)], axis=-1)
        z = jnp.maximum(gz_ref[...] + e_in @ wc_ref[...] + b1_ref[...], 0.0)
        o_ref[...] = jnp.sum(z * w2_ref[...], axis=-1, keepdims=True) + b2_ref[...]

    return pl.pallas_call(
        body,
        grid=(EE // EBT,),
        in_specs=[
            pl.BlockSpec((EBT, 64), lambda i: (i, 0)),
            pl.BlockSpec((NCH, EBT, CH), lambda i: (0, i, 0)),
            pl.BlockSpec((HID, 64), lambda i: (0, 0)),
            pl.BlockSpec((1, 64), lambda i: (0, 0)),
            pl.BlockSpec((1, 64), lambda i: (0, 0)),
            pl.BlockSpec((1, 1), lambda i: (0, 0)),
        ],
        out_specs=pl.BlockSpec((EBT, 1), lambda i: (i, 0)),
        out_shape=jax.ShapeDtypeStruct((EE, 1), jnp.float32),
    )(gz, e_st, w1c, b1[None, :], w2[None, :], b2[None, None, 0])


def kernel(x, e, edge_index, params):
    p = params
    src = edge_index[0]
    dst = edge_index[1]

    h = _tc_mlp(x, p['ne_W1'], p['ne_b1'], p['ne_W2'], p['ne_b2'], 2000)
    e_st = _tc_mlp_chunked_out(e, p['ee_W1'], p['ee_b1'], p['ee_W2'],
                               p['ee_b2'], EBT)

    w1 = p['pred_W1']
    for i in range(LAYERS):
        wcat = jnp.concatenate(
            [p['A1_W'][i], p['A2_W'][i], p['A3_W'][i],
             p['B1_W'][i], p['B2_W'][i]], axis=1)
        bcat = jnp.concatenate(
            [p['A1_b'][i], p['A2_b'][i], p['A3_b'][i],
             p['B1_b'][i], p['B2_b'][i]], axis=0)
        a1h, a2h, a3h, b1h, b2h = _tc_node_mm(h, wcat, bcat)
        gsum = _sc_gather2add(b1h, b2h, src, dst, HID)
        e_st = _tc_edge_update(gsum, e_st, p['B3_W'][i], p['B3_b'][i],
                               p['ln_e_g'][i], p['ln_e_b'][i])
        e_fl = e_st.reshape(NCH * EE, CH)
        f_fl, b_fl = _sc_agg(e_fl, a2h, a3h, src, dst)
        h = _tc_node_update(h, a1h, f_fl.reshape(2, NN, HID),
                            b_fl.reshape(2, NN, HID),
                            p['ln_h_g'][i], p['ln_h_b'][i])

    hab = _tc_pred_node(h, w1[0:2 * HID].reshape(2, HID, 64)
                        .transpose(1, 0, 2).reshape(HID, HID))
    gz = _sc_gather_head(hab, src, dst)
    return _tc_score_fin(gz, e_st, w1[2 * HID:3 * HID], p['pred_b1'],
                         p['pred_W2'][:, 0], p['pred_b2'])


# speedup vs baseline: 3.7073x; 1.0030x over previous
import functools

import jax
import jax.numpy as jnp
from jax import lax
from jax.experimental import pallas as pl
from jax.experimental.pallas import tpu as pltpu
from jax.experimental.pallas import tpu_sc as plsc

NN = 10000
EE = 320000
HID = 128
NCH = 2
CH = 64
LAYERS = 4
SC_CORES = 2
SC_TILES = 16

EBT = 2000


def _mesh():
    return plsc.VectorSubcoreMesh(
        core_axis_name="c", subcore_axis_name="s",
        num_cores=SC_CORES, num_subcores=SC_TILES)


def _sc_gather_kernel(ta, tb, src1d, dst1d, head):
    EB = 160
    HG = EB // 2
    SB = 4 * EB
    DO = 64 if head else HID

    scratch = [
        pltpu.VMEM((SB,), jnp.int32),
        pltpu.VMEM((SB,), jnp.int32),
        pltpu.VMEM((2, 2, HG), jnp.int32),
        pltpu.VMEM((2, 2, HG), jnp.int32),
        pltpu.VMEM((2, EB, HID), jnp.float32),
        pltpu.VMEM((2, EB, HID), jnp.float32),
        pltpu.SemaphoreType.DMA,
        pltpu.SemaphoreType.DMA,
        pltpu.SemaphoreType.DMA,
        pltpu.SemaphoreType.DMA,
        pltpu.SemaphoreType.DMA,
        pltpu.SemaphoreType.DMA,
    ]
    if head:
        scratch.insert(6, pltpu.VMEM((2, EB, 64), jnp.float32))

    @functools.partial(
        pl.kernel,
        out_type=jax.ShapeDtypeStruct((EE, DO), jnp.float32),
        mesh=_mesh(),
        scratch_types=scratch,
    )
    def k(ta_ref, tb_ref, s_ref, d_ref, out_ref, *scr):
        if head:
            ids1, idd1, iga, igb, abuf, bbuf, obuf = scr[:7]
            sems = scr[7:]
        else:
            ids1, idd1, iga, igb, abuf, bbuf = scr[:6]
            obuf = abuf
            sems = scr[6:]
        ga = sems[0:2]
        gb = sems[2:4]
        wr = sems[4:6]
        w = lax.axis_index("s") * SC_CORES + lax.axis_index("c")
        nblk = jnp.where(w == 31, 16, 64)
        wbase = w * 10240

        def stage_super(b0):
            pltpu.sync_copy(s_ref.at[pl.ds(wbase + b0 * EB, SB)], ids1)
            pltpu.sync_copy(d_ref.at[pl.ds(wbase + b0 * EB, SB)], idd1)

        def transform(b, slot):
            g = b % 4
            for q in range(2):
                for j in range(HG // 16):
                    jl = pl.ds(j * 16, 16)
                    fl = pl.ds(g * EB + q * HG + j * 16, 16)
                    iga[slot, q, jl] = ids1[fl]
                    igb[slot, q, jl] = idd1[fl]

        def issue(b, slot):
            for q in range(2):
                sl = pl.ds(q * HG, HG)
                pltpu.async_copy(ta_ref.at[iga.at[slot, q]],
                                 abuf.at[slot].at[sl], ga[slot])
                pltpu.async_copy(tb_ref.at[igb.at[slot, q]],
                                 bbuf.at[slot].at[sl], gb[slot])

        def wait_in(slot):
            for q in range(2):
                sl = pl.ds(q * HG, HG)
                pltpu.make_async_copy(ta_ref.at[iga.at[slot, q]],
                                      abuf.at[slot].at[sl], ga[slot]).wait()
                pltpu.make_async_copy(tb_ref.at[igb.at[slot, q]],
                                      bbuf.at[slot].at[sl], gb[slot]).wait()

        def wait_write(slot):
            pltpu.make_async_copy(obuf.at[slot],
                                  out_ref.at[pl.ds(0, EB)], wr[slot]).wait()

        stage_super(0)
        transform(0, 0)
        issue(0, 0)

        def body(b, carry):
            for par in range(2):

                @pl.when((b * 2 + par) < nblk)
                def _sub():
                    bb = b * 2 + par
                    wait_in(par)

                    def add_row(r, c):
                        if head:
                            for j in range(4):
                                obuf[par, r, pl.ds(j * 16, 16)] = (
                                    abuf[par, r, pl.ds(j * 16, 16)]
                                    + bbuf[par, r, pl.ds(64 + j * 16, 16)])
                        else:
                            for j in range(8):
                                jl = pl.ds(j * 16, 16)
                                abuf[par, r, jl] = (abuf[par, r, jl]
                                                    + bbuf[par, r, jl])
                        return c

                    lax.fori_loop(0, EB, add_row, 0)
                    pltpu.async_copy(
                        obuf.at[par],
                        out_ref.at[pl.ds(wbase + bb * EB, EB)], wr[par])
                    nxt = 1 - par

                    @pl.when(bb + 1 < nblk)
                    def _prefetch():
                        @pl.when(bb + 1 >= 2)
                        def _w1():
                            wait_write(nxt)

                        transform(bb + 1, nxt)
                        issue(bb + 1, nxt)

                    @pl.when(jnp.logical_and((bb + 2) % 4 == 0,
                                             bb + 2 < nblk))
                    def _stage():
                        stage_super(bb + 2)

            return carry

        lax.fori_loop(0, 32, body, 0)
        wait_write(0)
        wait_write(1)

    return k(ta, tb, src1d, dst1d)


def _sc_gather2add(ta, tb, src1d, dst1d, D):
    return _sc_gather_kernel(ta, tb, src1d, dst1d, head=False)


def _sc_gather_head(hab, src1d, dst1d):
    return _sc_gather_kernel(hab, hab, src1d, dst1d, head=True)


def _sc_agg(e_fl, a2, a3, src1d, dst1d):
    EB = 80
    SB = 8 * EB
    STRIPE = 624
    ZR = 8

    out2 = [jax.ShapeDtypeStruct((2 * NN, HID), jnp.float32)] * 2

    @functools.partial(
        pl.kernel,
        out_type=out2,
        mesh=_mesh(),
        scratch_types=[
            pltpu.VMEM((SB,), jnp.int32),
            pltpu.VMEM((SB,), jnp.int32),
            pltpu.VMEM((2, 1, EB), jnp.int32),
            pltpu.VMEM((2, 1, EB), jnp.int32),
            pltpu.VMEM((2, EB, 64), jnp.float32),
            pltpu.VMEM((2, EB, HID), jnp.float32),
            pltpu.VMEM((ZR, HID), jnp.float32),
            pltpu.VMEM_SHARED((NN, HID), jnp.float32),
            pltpu.SemaphoreType.DMA,
            pltpu.SemaphoreType.DMA,
            pltpu.SemaphoreType.DMA,
            pltpu.SemaphoreType.DMA,
            pltpu.SemaphoreType.DMA,
            pltpu.SemaphoreType.DMA,
        ],
    )
    def k(e_ref, a2_ref, a3_ref, s_ref, d_ref, f_o, b_o,
          ids1, idd1, idg, idc, ebuf, abuf, zbuf, acc,
          sg0, sg1, se0, se1, ss0, ss1, *_):
        t = lax.axis_index("s")
        core = lax.axis_index("c")
        is_last = t == SC_TILES - 1
        sgs = (sg0, sg1)
        ses = (se0, se1)
        sss = (ss0, ss1)

        def zrow(r, c):
            for j in range(8):
                zbuf[r, pl.ds(j * 16, 16)] = jnp.zeros((16,), jnp.float32)
            return c

        lax.fori_loop(0, ZR, zrow, 0)

        def zero_acc():
            nq = jnp.where(is_last, (STRIPE + 16) // ZR, STRIPE // ZR)

            def zq(q, c):
                pltpu.sync_copy(zbuf, acc.at[pl.ds(t * STRIPE + q * ZR, ZR)])
                return c

            lax.fori_loop(0, nq, zq, 0)

        def run_pass(h, fwd, tab_ref, out_ref):
            nblk = jnp.where(is_last, 160, 256)
            tbase = t * 20480

            def stage_super(b0):
                pltpu.sync_copy(s_ref.at[pl.ds(tbase + b0 * EB, SB)], ids1)
                pltpu.sync_copy(d_ref.at[pl.ds(tbase + b0 * EB, SB)], idd1)

            def transform(b, slot):
                g = b % 8
                gsrc, csrc = (ids1, idd1) if fwd else (idd1, ids1)
                for j in range(EB // 16):
                    jl = pl.ds(j * 16, 16)
                    fl = pl.ds(g * EB + j * 16, 16)
                    idg[slot, 0, jl] = gsrc[fl]
                    idc[slot, 0, jl] = csrc[fl]

            def issue(b, slot):
                pltpu.async_copy(tab_ref.at[idg.at[slot, 0]],
                                 abuf.at[slot], sgs[slot])
                pltpu.async_copy(e_ref.at[pl.ds(h * EE + tbase + b * EB, EB)],
                                 ebuf.at[slot], ses[slot])

            def wait_in(slot):
                pltpu.make_async_copy(tab_ref.at[idg.at[slot, 0]],
                                      abuf.at[slot], sgs[slot]).wait()
                pltpu.make_async_copy(e_ref.at[pl.ds(0, EB)],
                                      ebuf.at[slot], ses[slot]).wait()

            def wait_scatter(slot):
                pltpu.make_async_copy(abuf.at[slot],
                                      acc.at[idc.at[slot, 0]],
                                      sss[slot]).wait()

            stage_super(0)
            transform(0, 0)
            issue(0, 0)

            def body(b, carry):
                for par in range(2):

                    @pl.when((b * 2 + par) < nblk)
                    def _sub():
                        bb = b * 2 + par
                        wait_in(par)

                        def erow(r, c):
                            for j in range(4):
                                jl = pl.ds(j * 16, 16)
                                sg = 1.0 / (1.0 + jnp.exp(-ebuf[par, r, jl]))
                                av = abuf[par, r, pl.ds(h * 64 + j * 16, 16)]
                                abuf[par, r, jl] = sg * av
                                abuf[par, r, pl.ds(64 + j * 16, 16)] = sg
                            return c

                        lax.fori_loop(0, EB, erow, 0)
                        pltpu.async_copy(abuf.at[par],
                                         acc.at[idc.at[par, 0]],
                                         sss[par], add=True)

                        nxt = 1 - par

                        @pl.when(bb + 1 < nblk)
                        def _prefetch():
                            @pl.when(bb + 1 >= 2)
                            def _w1():
                                wait_scatter(nxt)

                            transform(bb + 1, nxt)
                            issue(bb + 1, nxt)

                        @pl.when(jnp.logical_and((bb + 2) % 8 == 0,
                                                 bb + 2 < nblk))
                        def _stage():
                            stage_super(bb + 2)

                return carry

            lax.fori_loop(0, 128, body, 0)
            wait_scatter(0)
            wait_scatter(1)
            plsc.subcore_barrier()
            pltpu.sync_copy(acc.at[pl.ds(t * STRIPE, STRIPE)],
                            out_ref.at[pl.ds(h * NN + t * STRIPE, STRIPE)])

            @pl.when(is_last)
            def _tail():
                pltpu.sync_copy(
                    acc.at[pl.ds(SC_TILES * STRIPE, NN - SC_TILES * STRIPE)],
                    out_ref.at[pl.ds(h * NN + SC_TILES * STRIPE,
                                     NN - SC_TILES * STRIPE)])

        zero_acc()
        plsc.subcore_barrier()
        for h in range(2):
            @pl.when(core == 0)
            def _fwd():
                run_pass(h, True, a2_ref, f_o)

            @pl.when(core == 1)
            def _bwd():
                run_pass(h, False, a3_ref, b_o)

            if h == 0:
                plsc.subcore_barrier()
                zero_acc()
                plsc.subcore_barrier()

    return k(e_fl, a2, a3, src1d, dst1d)


def _tc_mlp(x, w1, b1, w2, b2, block_rows):
    rows, din = x.shape
    dout = w2.shape[1]

    def body(x_ref, w1_ref, b1_ref, w2_ref, b2_ref, o_ref):
        hval = jnp.maximum(x_ref[...] @ w1_ref[...] + b1_ref[...], 0.0)
        o_ref[...] = hval @ w2_ref[...] + b2_ref[...]

    return pl.pallas_call(
        body,
        grid=(rows // block_rows,),
        in_specs=[
            pl.BlockSpec((block_rows, din), lambda i: (i, 0)),
            pl.BlockSpec(w1.shape, lambda i: (0, 0)),
            pl.BlockSpec((1, b1.shape[-1]), lambda i: (0, 0)),
            pl.BlockSpec(w2.shape, lambda i: (0, 0)),
            pl.BlockSpec((1, dout), lambda i: (0, 0)),
        ],
        out_specs=pl.BlockSpec((block_rows, dout), lambda i: (i, 0)),
        out_shape=jax.ShapeDtypeStruct((rows, dout), jnp.float32),
    )(x, w1, b1[None, :], w2, b2[None, :])


def _tc_mlp_chunked_out(x, w1, b1, w2, b2, block_rows):
    rows, din = x.shape

    def body(x_ref, w1_ref, b1_ref, w2_ref, b2_ref, o_ref):
        hval = jnp.maximum(x_ref[...] @ w1_ref[...] + b1_ref[...], 0.0)
        y = hval @ w2_ref[...] + b2_ref[...]
        for c in range(NCH):
            o_ref[c] = y[:, c * CH:(c + 1) * CH]

    return pl.pallas_call(
        body,
        grid=(rows // block_rows,),
        in_specs=[
            pl.BlockSpec((block_rows, din), lambda i: (i, 0)),
            pl.BlockSpec(w1.shape, lambda i: (0, 0)),
            pl.BlockSpec((1, b1.shape[-1]), lambda i: (0, 0)),
            pl.BlockSpec(w2.shape, lambda i: (0, 0)),
            pl.BlockSpec((1, HID), lambda i: (0, 0)),
        ],
        out_specs=pl.BlockSpec((NCH, block_rows, CH), lambda i: (0, i, 0)),
        out_shape=jax.ShapeDtypeStruct((NCH, rows, CH), jnp.float32),
    )(x, w1, b1[None, :], w2, b2[None, :])


def _tc_node_mm(h, wcat, bcat):
    block = 2000

    def body(h_ref, w_ref, b_ref, a1_ref, a2_ref, a3_ref, b1_ref, b2_ref):
        hw = h_ref[...] @ w_ref[...] + b_ref[...]
        a1_ref[...] = hw[:, 0:128]
        a2_ref[...] = hw[:, 128:256]
        a3_ref[...] = hw[:, 256:384]
        b1_ref[...] = hw[:, 384:512]
        b2_ref[...] = hw[:, 512:640]

    ospec = pl.BlockSpec((block, HID), lambda i: (i, 0))
    oshape = jax.ShapeDtypeStruct((NN, HID), jnp.float32)
    return pl.pallas_call(
        body,
        grid=(NN // block,),
        in_specs=[
            pl.BlockSpec((block, HID), lambda i: (i, 0)),
            pl.BlockSpec((HID, 5 * HID), lambda i: (0, 0)),
            pl.BlockSpec((1, 5 * HID), lambda i: (0, 0)),
        ],
        out_specs=[ospec] * 5,
        out_shape=[oshape] * 5,
    )(h, wcat, bcat[None, :])


def _tc_chunked_matmul(e_st, w, b):
    dout = w.shape[1]

    def body(e_ref, w_ref, b_ref, o_ref):
        x = jnp.concatenate([e_ref[c] for c in range(NCH)], axis=-1)
        o_ref[...] = x @ w_ref[...] + b_ref[...]

    return pl.pallas_call(
        body,
        grid=(EE // EBT,),
        in_specs=[
            pl.BlockSpec((NCH, EBT, CH), lambda i: (0, i, 0)),
            pl.BlockSpec((HID, dout), lambda i: (0, 0)),
            pl.BlockSpec((1, dout), lambda i: (0, 0)),
        ],
        out_specs=pl.BlockSpec((EBT, dout), lambda i: (i, 0)),
        out_shape=jax.ShapeDtypeStruct((EE, dout), jnp.float32),
    )(e_st, w, b[None, :])


def _layer_norm_rows(v, g, b):
    mu = jnp.mean(v, axis=-1, keepdims=True)
    var = jnp.mean((v - mu) ** 2, axis=-1, keepdims=True)
    return (v - mu) * jax.lax.rsqrt(var + 1e-5) * g + b


def _tc_edge_update(gsum, e_st, b3w, b3b, ln_g, ln_b):
    def body(g_ref, e_ref, w_ref, b_ref, lng_ref, lnb_ref, o_ref):
        e_in = jnp.concatenate([e_ref[c] for c in range(NCH)], axis=-1)
        b3e = e_in @ w_ref[...] + b_ref[...]
        e_hat = _layer_norm_rows(g_ref[...] + b3e, lng_ref[...], lnb_ref[...])
        e_new = e_in + jnp.maximum(e_hat, 0.0)
        for c in range(NCH):
            o_ref[c] = e_new[:, c * CH:(c + 1) * CH]

    return pl.pallas_call(
        body,
        grid=(EE // EBT,),
        in_specs=[
            pl.BlockSpec((EBT, HID), lambda i: (i, 0)),
            pl.BlockSpec((NCH, EBT, CH), lambda i: (0, i, 0)),
            pl.BlockSpec((HID, HID), lambda i: (0, 0)),
            pl.BlockSpec((1, HID), lambda i: (0, 0)),
            pl.BlockSpec((1, HID), lambda i: (0, 0)),
            pl.BlockSpec((1, HID), lambda i: (0, 0)),
        ],
        out_specs=pl.BlockSpec((NCH, EBT, CH), lambda i: (0, i, 0)),
        out_shape=jax.ShapeDtypeStruct((NCH, EE, CH), jnp.float32),
    )(gsum, e_st, b3w, b3b[None, :], ln_g[None, :], ln_b[None, :])


def _tc_node_update(h_in, a1h, f_st, b_st, ln_g, ln_b):
    block = 2000

    def body(h_ref, a1_ref, f_ref, b_ref, lng_ref, lnb_ref, o_ref):
        nf = jnp.concatenate([f_ref[0][:, 0:64], f_ref[1][:, 0:64]], axis=-1)
        df = jnp.concatenate([f_ref[0][:, 64:128], f_ref[1][:, 64:128]], axis=-1)
        nb = jnp.concatenate([b_ref[0][:, 0:64], b_ref[1][:, 0:64]], axis=-1)
        db = jnp.concatenate([b_ref[0][:, 64:128], b_ref[1][:, 64:128]], axis=-1)
        h_hat = a1_ref[...] + nf / (df + 1e-6) + nb / (db + 1e-6)
        h_hat = jnp.maximum(
            _layer_norm_rows(h_hat, lng_ref[...], lnb_ref[...]), 0.0)
        o_ref[...] = h_ref[...] + h_hat

    st = pl.BlockSpec((2, block, HID), lambda i: (0, i, 0))
    return pl.pallas_call(
        body,
        grid=(NN // block,),
        in_specs=[
            pl.BlockSpec((block, HID), lambda i: (i, 0)),
            pl.BlockSpec((block, HID), lambda i: (i, 0)),
            st, st,
            pl.BlockSpec((1, HID), lambda i: (0, 0)),
            pl.BlockSpec((1, HID), lambda i: (0, 0)),
        ],
        out_specs=pl.BlockSpec((block, HID), lambda i: (i, 0)),
        out_shape=jax.ShapeDtypeStruct((NN, HID), jnp.float32),
    )(h_in, a1h, f_st, b_st, ln_g[None, :], ln_b[None, :])


def _tc_pred_node(h, w1ab):
    block = 2000

    def body(h_ref, w_ref, o_ref):
        o_ref[...] = h_ref[...] @ w_ref[...]

    return pl.pallas_call(
        body,
        grid=(NN // block,),
        in_specs=[
            pl.BlockSpec((block, HID), lambda i: (i, 0)),
            pl.BlockSpec((HID, HID), lambda i: (0, 0)),
        ],
        out_specs=pl.BlockSpec((block, HID), lambda i: (i, 0)),
        out_shape=jax.ShapeDtypeStruct((NN, HID), jnp.float32),
    )(h, w1ab)


def _tc_score_fin(gz, e_st, w1c, b1, w2, b2):
    def body(gz_ref, e_ref, wc_ref, b1_ref, w2_ref, b2_ref, o_ref):
        e_in = jnp.concatenate([e_ref[c] for c in range(NCH)], axis=-1)
        z = jnp.maximum(gz_ref[...] + e_in @ wc_ref[...] + b1_ref[...], 0.0)
        o_ref[...] = jnp.sum(z * w2_ref[...], axis=-1, keepdims=True) + b2_ref[...]

    return pl.pallas_call(
        body,
        grid=(EE // EBT,),
        in_specs=[
            pl.BlockSpec((EBT, 64), lambda i: (i, 0)),
            pl.BlockSpec((NCH, EBT, CH), lambda i: (0, i, 0)),
            pl.BlockSpec((HID, 64), lambda i: (0, 0)),
            pl.BlockSpec((1, 64), lambda i: (0, 0)),
            pl.BlockSpec((1, 64), lambda i: (0, 0)),
            pl.BlockSpec((1, 1), lambda i: (0, 0)),
        ],
        out_specs=pl.BlockSpec((EBT, 1), lambda i: (i, 0)),
        out_shape=jax.ShapeDtypeStruct((EE, 1), jnp.float32),
    )(gz, e_st, w1c, b1[None, :], w2[None, :], b2[None, None, 0])


def kernel(x, e, edge_index, params):
    p = params
    src = edge_index[0]
    dst = edge_index[1]

    h = _tc_mlp(x, p['ne_W1'], p['ne_b1'], p['ne_W2'], p['ne_b2'], 2000)
    e_st = _tc_mlp_chunked_out(e, p['ee_W1'], p['ee_b1'], p['ee_W2'],
                               p['ee_b2'], EBT)

    w1 = p['pred_W1']
    for i in range(LAYERS):
        wcat = jnp.concatenate(
            [p['A1_W'][i], p['A2_W'][i], p['A3_W'][i],
             p['B1_W'][i], p['B2_W'][i]], axis=1)
        bcat = jnp.concatenate(
            [p['A1_b'][i], p['A2_b'][i], p['A3_b'][i],
             p['B1_b'][i], p['B2_b'][i]], axis=0)
        a1h, a2h, a3h, b1h, b2h = _tc_node_mm(h, wcat, bcat)
        gsum = _sc_gather2add(b1h, b2h, src, dst, HID)
        e_st = _tc_edge_update(gsum, e_st, p['B3_W'][i], p['B3_b'][i],
                               p['ln_e_g'][i], p['ln_e_b'][i])
        e_fl = e_st.reshape(NCH * EE, CH)
        f_fl, b_fl = _sc_agg(e_fl, a2h, a3h, src, dst)
        h = _tc_node_update(h, a1h, f_fl.reshape(2, NN, HID),
                            b_fl.reshape(2, NN, HID),
                            p['ln_h_g'][i], p['ln_h_b'][i])

    hab = _tc_pred_node(h, w1[0:2 * HID].reshape(2, HID, 64)
                        .transpose(1, 0, 2).reshape(HID, HID))
    gz = _sc_gather_head(hab, src, dst)
    return _tc_score_fin(gz, e_st, w1[2 * HID:3 * HID], p['pred_b1'],
                         p['pred_W2'][:, 0], p['pred_b2'])


# speedup vs baseline: 3.7141x; 1.0018x over previous
import functools

import jax
import jax.numpy as jnp
from jax import lax
from jax.experimental import pallas as pl
from jax.experimental.pallas import tpu as pltpu
from jax.experimental.pallas import tpu_sc as plsc

NN = 10000
EE = 320000
HID = 128
NCH = 2
CH = 64
LAYERS = 4
SC_CORES = 2
SC_TILES = 16

EBT = 2000


def _mesh():
    return plsc.VectorSubcoreMesh(
        core_axis_name="c", subcore_axis_name="s",
        num_cores=SC_CORES, num_subcores=SC_TILES)


def _sc_gather_kernel(ta, tb, src1d, dst1d, head):
    EB = 160
    HG = EB // 2
    SB = 4 * EB
    DO = 64 if head else HID

    scratch = [
        pltpu.VMEM((SB,), jnp.int32),
        pltpu.VMEM((SB,), jnp.int32),
        pltpu.VMEM((2, 2, HG), jnp.int32),
        pltpu.VMEM((2, 2, HG), jnp.int32),
        pltpu.VMEM((2, EB, HID), jnp.float32),
        pltpu.VMEM((2, EB, HID), jnp.float32),
        pltpu.SemaphoreType.DMA,
        pltpu.SemaphoreType.DMA,
        pltpu.SemaphoreType.DMA,
        pltpu.SemaphoreType.DMA,
        pltpu.SemaphoreType.DMA,
        pltpu.SemaphoreType.DMA,
    ]
    if head:
        scratch.insert(6, pltpu.VMEM((2, EB, 64), jnp.float32))

    @functools.partial(
        pl.kernel,
        out_type=jax.ShapeDtypeStruct((EE, DO), jnp.float32),
        mesh=_mesh(),
        scratch_types=scratch,
    )
    def k(ta_ref, tb_ref, s_ref, d_ref, out_ref, *scr):
        if head:
            ids1, idd1, iga, igb, abuf, bbuf, obuf = scr[:7]
            sems = scr[7:]
        else:
            ids1, idd1, iga, igb, abuf, bbuf = scr[:6]
            obuf = abuf
            sems = scr[6:]
        ga = sems[0:2]
        gb = sems[2:4]
        wr = sems[4:6]
        w = lax.axis_index("s") * SC_CORES + lax.axis_index("c")
        nblk = jnp.where(w == 31, 16, 64)
        wbase = w * 10240

        def stage_super(b0):
            pltpu.sync_copy(s_ref.at[pl.ds(wbase + b0 * EB, SB)], ids1)
            pltpu.sync_copy(d_ref.at[pl.ds(wbase + b0 * EB, SB)], idd1)

        def transform(b, slot):
            g = b % 4
            for q in range(2):
                for j in range(HG // 16):
                    jl = pl.ds(j * 16, 16)
                    fl = pl.ds(g * EB + q * HG + j * 16, 16)
                    iga[slot, q, jl] = ids1[fl]
                    igb[slot, q, jl] = idd1[fl]

        def issue(b, slot):
            for q in range(2):
                sl = pl.ds(q * HG, HG)
                pltpu.async_copy(ta_ref.at[iga.at[slot, q]],
                                 abuf.at[slot].at[sl], ga[slot])
                pltpu.async_copy(tb_ref.at[igb.at[slot, q]],
                                 bbuf.at[slot].at[sl], gb[slot])

        def wait_in(slot):
            for q in range(2):
                sl = pl.ds(q * HG, HG)
                pltpu.make_async_copy(ta_ref.at[iga.at[slot, q]],
                                      abuf.at[slot].at[sl], ga[slot]).wait()
                pltpu.make_async_copy(tb_ref.at[igb.at[slot, q]],
                                      bbuf.at[slot].at[sl], gb[slot]).wait()

        def wait_write(slot):
            pltpu.make_async_copy(obuf.at[slot],
                                  out_ref.at[pl.ds(0, EB)], wr[slot]).wait()

        stage_super(0)
        transform(0, 0)
        issue(0, 0)

        def body(b, carry):
            for par in range(2):

                @pl.when((b * 2 + par) < nblk)
                def _sub():
                    bb = b * 2 + par
                    wait_in(par)

                    def add_row(r, c):
                        if head:
                            for j in range(4):
                                obuf[par, r, pl.ds(j * 16, 16)] = (
                                    abuf[par, r, pl.ds(j * 16, 16)]
                                    + bbuf[par, r, pl.ds(64 + j * 16, 16)])
                        else:
                            for j in range(8):
                                jl = pl.ds(j * 16, 16)
                                abuf[par, r, jl] = (abuf[par, r, jl]
                                                    + bbuf[par, r, jl])
                        return c

                    lax.fori_loop(0, EB, add_row, 0)
                    pltpu.async_copy(
                        obuf.at[par],
                        out_ref.at[pl.ds(wbase + bb * EB, EB)], wr[par])
                    nxt = 1 - par

                    @pl.when(bb + 1 < nblk)
                    def _prefetch():
                        @pl.when(bb + 1 >= 2)
                        def _w1():
                            wait_write(nxt)

                        transform(bb + 1, nxt)
                        issue(bb + 1, nxt)

                    @pl.when(jnp.logical_and((bb + 2) % 4 == 0,
                                             bb + 2 < nblk))
                    def _stage():
                        stage_super(bb + 2)

            return carry

        lax.fori_loop(0, 32, body, 0)
        wait_write(0)
        wait_write(1)

    return k(ta, tb, src1d, dst1d)


def _sc_gather2add(ta, tb, src1d, dst1d, D):
    return _sc_gather_kernel(ta, tb, src1d, dst1d, head=False)


def _sc_gather_head(hab, src1d, dst1d):
    return _sc_gather_kernel(hab, hab, src1d, dst1d, head=True)


def _sc_agg(e_fl, a2, a3, src1d, dst1d):
    EB = 80
    SB = 8 * EB
    STRIPE = 624
    ZR = 8

    out2 = [jax.ShapeDtypeStruct((2 * NN, HID), jnp.float32)] * 2

    @functools.partial(
        pl.kernel,
        out_type=out2,
        mesh=_mesh(),
        scratch_types=[
            pltpu.VMEM((SB,), jnp.int32),
            pltpu.VMEM((SB,), jnp.int32),
            pltpu.VMEM((2, 1, EB), jnp.int32),
            pltpu.VMEM((2, 1, EB), jnp.int32),
            pltpu.VMEM((2, EB, 64), jnp.float32),
            pltpu.VMEM((2, EB, HID), jnp.float32),
            pltpu.VMEM((ZR, HID), jnp.float32),
            pltpu.VMEM_SHARED((NN, HID), jnp.float32),
            pltpu.SemaphoreType.DMA,
            pltpu.SemaphoreType.DMA,
            pltpu.SemaphoreType.DMA,
            pltpu.SemaphoreType.DMA,
            pltpu.SemaphoreType.DMA,
            pltpu.SemaphoreType.DMA,
        ],
    )
    def k(e_ref, a2_ref, a3_ref, s_ref, d_ref, f_o, b_o,
          ids1, idd1, idg, idc, ebuf, abuf, zbuf, acc,
          sg0, sg1, se0, se1, ss0, ss1, *_):
        t = lax.axis_index("s")
        core = lax.axis_index("c")
        is_last = t == SC_TILES - 1
        sgs = (sg0, sg1)
        ses = (se0, se1)
        sss = (ss0, ss1)

        def zrow(r, c):
            for j in range(8):
                zbuf[r, pl.ds(j * 16, 16)] = jnp.zeros((16,), jnp.float32)
            return c

        lax.fori_loop(0, ZR, zrow, 0)

        def zero_acc():
            nq = jnp.where(is_last, (STRIPE + 16) // ZR, STRIPE // ZR)

            def zq(q, c):
                pltpu.sync_copy(zbuf, acc.at[pl.ds(t * STRIPE + q * ZR, ZR)])
                return c

            lax.fori_loop(0, nq, zq, 0)

        def run_pass(h, fwd, tab_ref, out_ref):
            nblk = jnp.where(is_last, 160, 256)
            tbase = t * 20480

            def stage_super(b0):
                pltpu.sync_copy(s_ref.at[pl.ds(tbase + b0 * EB, SB)], ids1)
                pltpu.sync_copy(d_ref.at[pl.ds(tbase + b0 * EB, SB)], idd1)

            def transform(b, slot):
                g = b % 8
                gsrc, csrc = (ids1, idd1) if fwd else (idd1, ids1)
                for j in range(EB // 16):
                    jl = pl.ds(j * 16, 16)
                    fl = pl.ds(g * EB + j * 16, 16)
                    idg[slot, 0, jl] = gsrc[fl]
                    idc[slot, 0, jl] = csrc[fl]

            def issue(b, slot):
                pltpu.async_copy(tab_ref.at[idg.at[slot, 0]],
                                 abuf.at[slot], sgs[slot])
                pltpu.async_copy(e_ref.at[pl.ds(h * EE + tbase + b * EB, EB)],
                                 ebuf.at[slot], ses[slot])

            def wait_in(slot):
                pltpu.make_async_copy(tab_ref.at[idg.at[slot, 0]],
                                      abuf.at[slot], sgs[slot]).wait()
                pltpu.make_async_copy(e_ref.at[pl.ds(0, EB)],
                                      ebuf.at[slot], ses[slot]).wait()

            def wait_scatter(slot):
                pltpu.make_async_copy(abuf.at[slot],
                                      acc.at[idc.at[slot, 0]],
                                      sss[slot]).wait()

            stage_super(0)
            transform(0, 0)
            issue(0, 0)

            def body(b, carry):
                for par in range(2):

                    @pl.when((b * 2 + par) < nblk)
                    def _sub():
                        bb = b * 2 + par
                        wait_in(par)

                        def erow(r, c):
                            for j in range(4):
                                jl = pl.ds(j * 16, 16)
                                sg = 1.0 / (1.0 + jnp.exp(-ebuf[par, r, jl]))
                                av = abuf[par, r, pl.ds(h * 64 + j * 16, 16)]
                                abuf[par, r, jl] = sg * av
                                abuf[par, r, pl.ds(64 + j * 16, 16)] = sg
                            return c

                        lax.fori_loop(0, EB, erow, 0)
                        pltpu.async_copy(abuf.at[par],
                                         acc.at[idc.at[par, 0]],
                                         sss[par], add=True)

                        nxt = 1 - par

                        @pl.when(bb + 1 < nblk)
                        def _prefetch():
                            @pl.when(bb + 1 >= 2)
                            def _w1():
                                wait_scatter(nxt)

                            transform(bb + 1, nxt)
                            issue(bb + 1, nxt)

                        @pl.when(jnp.logical_and((bb + 2) % 8 == 0,
                                                 bb + 2 < nblk))
                        def _stage():
                            stage_super(bb + 2)

                return carry

            lax.fori_loop(0, 128, body, 0)
            wait_scatter(0)
            wait_scatter(1)
            plsc.subcore_barrier()
            pltpu.sync_copy(acc.at[pl.ds(t * STRIPE, STRIPE)],
                            out_ref.at[pl.ds(h * NN + t * STRIPE, STRIPE)])

            @pl.when(is_last)
            def _tail():
                pltpu.sync_copy(
                    acc.at[pl.ds(SC_TILES * STRIPE, NN - SC_TILES * STRIPE)],
                    out_ref.at[pl.ds(h * NN + SC_TILES * STRIPE,
                                     NN - SC_TILES * STRIPE)])

        zero_acc()
        plsc.subcore_barrier()
        for h in range(2):
            @pl.when(core == 0)
            def _fwd():
                run_pass(h, True, a2_ref, f_o)

            @pl.when(core == 1)
            def _bwd():
                run_pass(h, False, a3_ref, b_o)

            if h == 0:
                plsc.subcore_barrier()
                zero_acc()
                plsc.subcore_barrier()

    return k(e_fl, a2, a3, src1d, dst1d)


def _tc_mlp(x, w1, b1, w2, b2, block_rows):
    rows, din = x.shape
    dout = w2.shape[1]

    def body(x_ref, w1_ref, b1_ref, w2_ref, b2_ref, o_ref):
        hval = jnp.maximum(x_ref[...] @ w1_ref[...] + b1_ref[...], 0.0)
        o_ref[...] = hval @ w2_ref[...] + b2_ref[...]

    return pl.pallas_call(
        body,
        grid=(rows // block_rows,),
        in_specs=[
            pl.BlockSpec((block_rows, din), lambda i: (i, 0)),
            pl.BlockSpec(w1.shape, lambda i: (0, 0)),
            pl.BlockSpec((1, b1.shape[-1]), lambda i: (0, 0)),
            pl.BlockSpec(w2.shape, lambda i: (0, 0)),
            pl.BlockSpec((1, dout), lambda i: (0, 0)),
        ],
        out_specs=pl.BlockSpec((block_rows, dout), lambda i: (i, 0)),
        out_shape=jax.ShapeDtypeStruct((rows, dout), jnp.float32),
    )(x, w1, b1[None, :], w2, b2[None, :])


def _tc_mlp_chunked_out(x, w1, b1, w2, b2, block_rows):
    rows, din = x.shape

    def body(x_ref, w1_ref, b1_ref, w2_ref, b2_ref, o_ref):
        hval = jnp.maximum(x_ref[...] @ w1_ref[...] + b1_ref[...], 0.0)
        y = hval @ w2_ref[...] + b2_ref[...]
        for c in range(NCH):
            o_ref[c] = y[:, c * CH:(c + 1) * CH]

    return pl.pallas_call(
        body,
        grid=(rows // block_rows,),
        in_specs=[
            pl.BlockSpec((block_rows, din), lambda i: (i, 0)),
            pl.BlockSpec(w1.shape, lambda i: (0, 0)),
            pl.BlockSpec((1, b1.shape[-1]), lambda i: (0, 0)),
            pl.BlockSpec(w2.shape, lambda i: (0, 0)),
            pl.BlockSpec((1, HID), lambda i: (0, 0)),
        ],
        out_specs=pl.BlockSpec((NCH, block_rows, CH), lambda i: (0, i, 0)),
        out_shape=jax.ShapeDtypeStruct((NCH, rows, CH), jnp.float32),
    )(x, w1, b1[None, :], w2, b2[None, :])


def _tc_node_mm(h, wcat, bcat):
    block = 2000

    def body(h_ref, w_ref, b_ref, a1_ref, a2_ref, a3_ref, b1_ref, b2_ref):
        hw = h_ref[...] @ w_ref[...] + b_ref[...]
        a1_ref[...] = hw[:, 0:128]
        a2_ref[...] = hw[:, 128:256]
        a3_ref[...] = hw[:, 256:384]
        b1_ref[...] = hw[:, 384:512]
        b2_ref[...] = hw[:, 512:640]

    ospec = pl.BlockSpec((block, HID), lambda i: (i, 0))
    oshape = jax.ShapeDtypeStruct((NN, HID), jnp.float32)
    return pl.pallas_call(
        body,
        grid=(NN // block,),
        in_specs=[
            pl.BlockSpec((block, HID), lambda i: (i, 0)),
            pl.BlockSpec((HID, 5 * HID), lambda i: (0, 0)),
            pl.BlockSpec((1, 5 * HID), lambda i: (0, 0)),
        ],
        out_specs=[ospec] * 5,
        out_shape=[oshape] * 5,
    )(h, wcat, bcat[None, :])


def _tc_chunked_matmul(e_st, w, b):
    dout = w.shape[1]

    def body(e_ref, w_ref, b_ref, o_ref):
        x = jnp.concatenate([e_ref[c] for c in range(NCH)], axis=-1)
        o_ref[...] = x @ w_ref[...] + b_ref[...]

    return pl.pallas_call(
        body,
        grid=(EE // EBT,),
        in_specs=[
            pl.BlockSpec((NCH, EBT, CH), lambda i: (0, i, 0)),
            pl.BlockSpec((HID, dout), lambda i: (0, 0)),
            pl.BlockSpec((1, dout), lambda i: (0, 0)),
        ],
        out_specs=pl.BlockSpec((EBT, dout), lambda i: (i, 0)),
        out_shape=jax.ShapeDtypeStruct((EE, dout), jnp.float32),
    )(e_st, w, b[None, :])


def _layer_norm_rows(v, g, b):
    mu = jnp.mean(v, axis=-1, keepdims=True)
    var = jnp.mean((v - mu) ** 2, axis=-1, keepdims=True)
    return (v - mu) * jax.lax.rsqrt(var + 1e-5) * g + b


def _tc_edge_update(gsum, e_st, b3w, b3b, ln_g, ln_b):
    def body(g_ref, e_ref, w_ref, b_ref, lng_ref, lnb_ref, o_ref):
        e_in = jnp.concatenate([e_ref[c] for c in range(NCH)], axis=-1)
        b3e = e_in @ w_ref[...] + b_ref[...]
        e_hat = _layer_norm_rows(g_ref[...] + b3e, lng_ref[...], lnb_ref[...])
        e_new = e_in + jnp.maximum(e_hat, 0.0)
        for c in range(NCH):
            o_ref[c] = e_new[:, c * CH:(c + 1) * CH]

    return pl.pallas_call(
        body,
        grid=(EE // EBT,),
        in_specs=[
            pl.BlockSpec((EBT, HID), lambda i: (i, 0)),
            pl.BlockSpec((NCH, EBT, CH), lambda i: (0, i, 0)),
            pl.BlockSpec((HID, HID), lambda i: (0, 0)),
            pl.BlockSpec((1, HID), lambda i: (0, 0)),
            pl.BlockSpec((1, HID), lambda i: (0, 0)),
            pl.BlockSpec((1, HID), lambda i: (0, 0)),
        ],
        out_specs=pl.BlockSpec((NCH, EBT, CH), lambda i: (0, i, 0)),
        out_shape=jax.ShapeDtypeStruct((NCH, EE, CH), jnp.float32),
    )(gsum, e_st, b3w, b3b[None, :], ln_g[None, :], ln_b[None, :])


def _tc_node_update(h_in, a1h, f_st, b_st, ln_g, ln_b):
    block = 2000

    def body(h_ref, a1_ref, f_ref, b_ref, lng_ref, lnb_ref, o_ref):
        nf = jnp.concatenate([f_ref[0][:, 0:64], f_ref[1][:, 0:64]], axis=-1)
        df = jnp.concatenate([f_ref[0][:, 64:128], f_ref[1][:, 64:128]], axis=-1)
        nb = jnp.concatenate([b_ref[0][:, 0:64], b_ref[1][:, 0:64]], axis=-1)
        db = jnp.concatenate([b_ref[0][:, 64:128], b_ref[1][:, 64:128]], axis=-1)
        h_hat = a1_ref[...] + nf / (df + 1e-6) + nb / (db + 1e-6)
        h_hat = jnp.maximum(
            _layer_norm_rows(h_hat, lng_ref[...], lnb_ref[...]), 0.0)
        o_ref[...] = h_ref[...] + h_hat

    st = pl.BlockSpec((2, block, HID), lambda i: (0, i, 0))
    return pl.pallas_call(
        body,
        grid=(NN // block,),
        in_specs=[
            pl.BlockSpec((block, HID), lambda i: (i, 0)),
            pl.BlockSpec((block, HID), lambda i: (i, 0)),
            st, st,
            pl.BlockSpec((1, HID), lambda i: (0, 0)),
            pl.BlockSpec((1, HID), lambda i: (0, 0)),
        ],
        out_specs=pl.BlockSpec((block, HID), lambda i: (i, 0)),
        out_shape=jax.ShapeDtypeStruct((NN, HID), jnp.float32),
    )(h_in, a1h, f_st, b_st, ln_g[None, :], ln_b[None, :])


def _tc_pred_node(h, w1ab):
    block = 2000

    def body(h_ref, w_ref, o_ref):
        o_ref[...] = h_ref[...] @ w_ref[...]

    return pl.pallas_call(
        body,
        grid=(NN // block,),
        in_specs=[
            pl.BlockSpec((block, HID), lambda i: (i, 0)),
            pl.BlockSpec((HID, HID), lambda i: (0, 0)),
        ],
        out_specs=pl.BlockSpec((block, HID), lambda i: (i, 0)),
        out_shape=jax.ShapeDtypeStruct((NN, HID), jnp.float32),
    )(h, w1ab)


def _tc_score_fin(gz, e_st, w1c, b1, w2, b2):
    def body(gz_ref, e_ref, wc_ref, b1_ref, w2_ref, b2_ref, o_ref):
        e_in = jnp.concatenate([e_ref[c] for c in range(NCH)], axis=-1)
        z = jnp.maximum(gz_ref[...] + e_in @ wc_ref[...] + b1_ref[...], 0.0)
        o_ref[...] = jnp.sum(z * w2_ref[...], axis=-1, keepdims=True) + b2_ref[...]

    return pl.pallas_call(
        body,
        grid=(EE // EBT,),
        in_specs=[
            pl.BlockSpec((EBT, 64), lambda i: (i, 0)),
            pl.BlockSpec((NCH, EBT, CH), lambda i: (0, i, 0)),
            pl.BlockSpec((HID, 64), lambda i: (0, 0)),
            pl.BlockSpec((1, 64), lambda i: (0, 0)),
            pl.BlockSpec((1, 64), lambda i: (0, 0)),
            pl.BlockSpec((1, 1), lambda i: (0, 0)),
        ],
        out_specs=pl.BlockSpec((EBT, 1), lambda i: (i, 0)),
        out_shape=jax.ShapeDtypeStruct((EE, 1), jnp.float32),
    )(gz, e_st, w1c, b1[None, :], w2[None, :], b2[None, None, 0])


def kernel(x, e, edge_index, params):
    p = params
    src = edge_index[0]
    dst = edge_index[1]

    h = _tc_mlp(x, p['ne_W1'], p['ne_b1'], p['ne_W2'], p['ne_b2'], 2000)
    e_st = None

    w1 = p['pred_W1']
    for i in range(LAYERS):
        wcat = jnp.concatenate(
            [p['A1_W'][i], p['A2_W'][i], p['A3_W'][i],
             p['B1_W'][i], p['B2_W'][i]], axis=1)
        bcat = jnp.concatenate(
            [p['A1_b'][i], p['A2_b'][i], p['A3_b'][i],
             p['B1_b'][i], p['B2_b'][i]], axis=0)
        a1h, a2h, a3h, b1h, b2h = _tc_node_mm(h, wcat, bcat)
        gsum = _sc_gather2add(b1h, b2h, src, dst, HID)
        if i == 0:
            e_st = _tc_mlp_chunked_out(e, p['ee_W1'], p['ee_b1'],
                                       p['ee_W2'], p['ee_b2'], EBT)
        e_st = _tc_edge_update(gsum, e_st, p['B3_W'][i], p['B3_b'][i],
                               p['ln_e_g'][i], p['ln_e_b'][i])
        e_fl = e_st.reshape(NCH * EE, CH)
        f_fl, b_fl = _sc_agg(e_fl, a2h, a3h, src, dst)
        h = _tc_node_update(h, a1h, f_fl.reshape(2, NN, HID),
                            b_fl.reshape(2, NN, HID),
                            p['ln_h_g'][i], p['ln_h_b'][i])

    hab = _tc_pred_node(h, w1[0:2 * HID].reshape(2, HID, 64)
                        .transpose(1, 0, 2).reshape(HID, HID))
    gz = _sc_gather_head(hab, src, dst)
    return _tc_score_fin(gz, e_st, w1[2 * HID:3 * HID], p['pred_b1'],
                         p['pred_W2'][:, 0], p['pred_b2'])
